# baseline (speedup 1.0000x reference)
import jax
import jax.numpy as jnp
import numpy as np
from jax.experimental import pallas as pl

LATENT = 128
RADIUS = 0.015
BOUNDS = np.array([[0.1, 0.9], [0.1, 0.9], [0.1, 0.9]], dtype=np.float32)


def _mlp(ps, x):
    for i, p in enumerate(ps):
        x = x @ p["W"] + p["b"]
        if i < len(ps) - 1:
            x = jax.nn.relu(x)
    return x


def _ln(x, p):
    m = jnp.mean(x, axis=-1, keepdims=True)
    v = jnp.var(x, axis=-1, keepdims=True)
    return (x - m) / jnp.sqrt(v + 1e-5) * p["g"] + p["b"]


def _id_body(x_ref, o_ref):
    o_ref[...] = x_ref[...]


def kernel(position_sequence, particle_types, edge_index, n_particles_per_example, params):
    boundaries = jnp.asarray(BOUNDS)
    most_recent = position_sequence[:, -1]
    vel = position_sequence[:, 1:] - position_sequence[:, :-1]
    flat_vel = vel.reshape(position_sequence.shape[0], -1)
    d_lo = most_recent - boundaries[:, 0][None]
    d_hi = boundaries[:, 1][None] - most_recent
    d_b = jnp.clip(jnp.concatenate([d_lo, d_hi], axis=1) / RADIUS, -1.0, 1.0)
    temb = params["type_emb"][particle_types]
    x = jnp.concatenate([flat_vel, d_b, temb], axis=-1)
    senders = edge_index[0]
    receivers = edge_index[1]
    rel = (most_recent[senders, :] - most_recent[receivers, :]) / RADIUS
    dist = jnp.linalg.norm(rel, axis=-1, keepdims=True)
    e = jnp.concatenate([rel, dist], axis=-1)
    h = _ln(_mlp(params["node_enc"], x), params["node_enc_ln"])
    he = _ln(_mlp(params["edge_enc"], e), params["edge_enc_ln"])
    nnodes = h.shape[0]
    for sp in params["steps"]:
        ein = jnp.concatenate([h[senders], h[receivers], he], axis=-1)
        he_new = _ln(_mlp(sp["edge_mlp"], ein), sp["edge_ln"])
        agg = jax.ops.segment_sum(he_new, receivers, num_segments=nnodes)
        nin = jnp.concatenate([h, agg], axis=-1)
        h_new = _ln(_mlp(sp["node_mlp"], nin), sp["node_ln"])
        h = h + h_new
        he = he + he_new
    h = pl.pallas_call(_id_body, out_shape=jax.ShapeDtypeStruct(h.shape, h.dtype))(h)
    acc = _mlp(params["decoder"], h)
    prev_vel = position_sequence[:, -1] - position_sequence[:, -2]
    new_vel = prev_vel + acc
    new_pos = position_sequence[:, -1] + new_vel
    return new_pos


# speedup vs baseline: 2.8335x; 2.8335x over previous
import functools

import jax
import jax.numpy as jnp
import numpy as np
from jax import lax
from jax.experimental import pallas as pl
from jax.experimental.pallas import tpu as pltpu
from jax.experimental.pallas import tpu_sc as plsc

RADIUS = 0.015
_BOUNDS = np.array([[0.1, 0.9], [0.1, 0.9], [0.1, 0.9]], dtype=np.float32)
_EPS = 1e-5
_NC, _NS = 2, 16
_NW = _NC * _NS
_C = 40
_BN = 1000
_BE = 2560
_L = 128


def _pair_gather(n_edges, d, sign):
    ew = n_edges // _NW
    nch = ew // _C
    assert ew * _NW == n_edges and nch * _C == ew and nch % 2 == 0
    mesh = plsc.VectorSubcoreMesh(core_axis_name="c", subcore_axis_name="s")

    def body(a_hbm, b_hbm, s_hbm, r_hbm, out_hbm,
             si0, ri0, si1, ri1, bp0, bq0, bp1, bq1, sa0, sb0, sa1, sb1):
        wid = lax.axis_index("s") * _NC + lax.axis_index("c")
        base = pl.multiple_of(wid * ew, 8)

        def load_idx(j, si, ri):
            off = pl.multiple_of(base + j * _C, 8)
            pltpu.sync_copy(s_hbm.at[pl.ds(off, _C)], si)
            pltpu.sync_copy(r_hbm.at[pl.ds(off, _C)], ri)

        def fire(si, ri, bp, bq, sa, sb):
            pltpu.async_copy(a_hbm.at[si], bp, sa)
            pltpu.async_copy(b_hbm.at[ri], bq, sb)

        def wait(si, ri, bp, bq, sa, sb):
            pltpu.make_async_copy(a_hbm.at[si], bp, sa).wait()
            pltpu.make_async_copy(b_hbm.at[ri], bq, sb).wait()

        def combine_store(j, bp, bq):
            def row(i, c):
                for g in range(d // 16):
                    sl = pl.ds(g * 16, 16)
                    if sign > 0:
                        bp[i, sl] = bp[i, sl] + bq[i, sl]
                    else:
                        bp[i, sl] = bp[i, sl] - bq[i, sl]
                return c
            lax.fori_loop(0, _C, row, 0)
            off = pl.multiple_of(base + j * _C, 8)
            pltpu.sync_copy(bp, out_hbm.at[pl.ds(off, _C)])

        load_idx(0, si0, ri0)
        fire(si0, ri0, bp0, bq0, sa0, sb0)
        load_idx(1, si1, ri1)
        fire(si1, ri1, bp1, bq1, sa1, sb1)

        def pair(k, c):
            j0 = 2 * k
            wait(si0, ri0, bp0, bq0, sa0, sb0)
            combine_store(j0, bp0, bq0)
            load_idx(j0 + 2, si0, ri0)
            fire(si0, ri0, bp0, bq0, sa0, sb0)
            wait(si1, ri1, bp1, bq1, sa1, sb1)
            combine_store(j0 + 1, bp1, bq1)
            load_idx(j0 + 3, si1, ri1)
            fire(si1, ri1, bp1, bq1, sa1, sb1)
            return c

        lax.fori_loop(0, nch // 2 - 1, pair, 0)
        wait(si0, ri0, bp0, bq0, sa0, sb0)
        combine_store(nch - 2, bp0, bq0)
        wait(si1, ri1, bp1, bq1, sa1, sb1)
        combine_store(nch - 1, bp1, bq1)

    return functools.partial(
        pl.kernel, body,
        out_type=jax.ShapeDtypeStruct((n_edges, d), jnp.float32),
        mesh=mesh,
        compiler_params=pltpu.CompilerParams(use_tc_tiling_on_sc=(d % 128 == 0)),
        scratch_types=[
            pltpu.VMEM((_C,), jnp.int32), pltpu.VMEM((_C,), jnp.int32),
            pltpu.VMEM((_C,), jnp.int32), pltpu.VMEM((_C,), jnp.int32),
            pltpu.VMEM((_C, d), jnp.float32), pltpu.VMEM((_C, d), jnp.float32),
            pltpu.VMEM((_C, d), jnp.float32), pltpu.VMEM((_C, d), jnp.float32),
            pltpu.SemaphoreType.DMA, pltpu.SemaphoreType.DMA,
            pltpu.SemaphoreType.DMA, pltpu.SemaphoreType.DMA,
        ],
    )()


def _scatter_add(n_nodes, n_edges, d):
    ew = n_edges // _NW
    nch = ew // _C
    rt = (-(-n_nodes // _NS) + 7) // 8 * 8
    n_pad = rt * _NS
    last = n_nodes - rt * (_NS - 1)
    assert nch * _C == ew and nch % 2 == 0
    assert last > 0 and last % 8 == 0 and n_nodes % 8 == 0
    mesh = plsc.VectorSubcoreMesh(core_axis_name="c", subcore_axis_name="s")

    def body(v_hbm, r_hbm, z_hbm, out_hbm, ri0, ri1, b0, b1, sa0, sa1, shared):
        cid = lax.axis_index("c")
        sid = lax.axis_index("s")
        wid = sid * _NC + cid
        base = pl.multiple_of(wid * ew, 8)
        roff = pl.multiple_of(sid * rt, 8)

        pltpu.sync_copy(z_hbm, shared.at[pl.ds(roff, rt)])
        plsc.subcore_barrier()

        def load(j, ri, b, sa):
            off = pl.multiple_of(base + j * _C, 8)
            pltpu.sync_copy(r_hbm.at[pl.ds(off, _C)], ri)
            pltpu.async_copy(v_hbm.at[pl.ds(off, _C)], b, sa)

        def wait(b, sa):
            pltpu.make_async_copy(v_hbm.at[pl.ds(0, _C)], b, sa).wait()

        load(0, ri0, b0, sa0)
        load(1, ri1, b1, sa1)

        def pair(k, c):
            j0 = 2 * k
            wait(b0, sa0)
            pltpu.sync_copy(b0, shared.at[ri0], add=True)
            load(j0 + 2, ri0, b0, sa0)
            wait(b1, sa1)
            pltpu.sync_copy(b1, shared.at[ri1], add=True)
            load(j0 + 3, ri1, b1, sa1)
            return c

        lax.fori_loop(0, nch // 2 - 1, pair, 0)
        wait(b0, sa0)
        pltpu.sync_copy(b0, shared.at[ri0], add=True)
        wait(b1, sa1)
        pltpu.sync_copy(b1, shared.at[ri1], add=True)

        plsc.subcore_barrier()
        obase = pl.multiple_of(cid * n_nodes + roff, 8)

        @pl.when(sid == _NS - 1)
        def _dump_last():
            pltpu.sync_copy(shared.at[pl.ds(roff, last)],
                            out_hbm.at[pl.ds(obase, last)])

        @pl.when(sid < _NS - 1)
        def _dump_full():
            pltpu.sync_copy(shared.at[pl.ds(roff, rt)],
                            out_hbm.at[pl.ds(obase, rt)])

    return functools.partial(
        pl.kernel, body,
        out_type=jax.ShapeDtypeStruct((2 * n_nodes, d), jnp.float32),
        mesh=mesh,
        scratch_types=[
            pltpu.VMEM((_C,), jnp.int32), pltpu.VMEM((_C,), jnp.int32),
            pltpu.VMEM((_C, d), jnp.float32), pltpu.VMEM((_C, d), jnp.float32),
            pltpu.SemaphoreType.DMA, pltpu.SemaphoreType.DMA,
            pltpu.VMEM_SHARED((n_pad, d), jnp.float32),
        ],
    )()


def _ln_in(v, g, b):
    m = jnp.mean(v, axis=-1, keepdims=True)
    var = jnp.mean((v - m) ** 2, axis=-1, keepdims=True)
    return (v - m) / jnp.sqrt(var + _EPS) * g + b


def _dot(a, b):
    return jnp.dot(a, b, preferred_element_type=jnp.float32)


def _wspec():
    return pl.BlockSpec((_L, _L), lambda i: (0, 0))


def _bspec():
    return pl.BlockSpec((1, _L), lambda i: (0, 0))


def _node_enc_body(x_ref, w1, b1, w2, b2, w3, b3, g, bb, wpq,
                   h_ref, p_ref, q_ref):
    t = jnp.maximum(_dot(x_ref[...], w1[...]) + b1[...], 0.0)
    t = jnp.maximum(_dot(t, w2[...]) + b2[...], 0.0)
    v = _dot(t, w3[...]) + b3[...]
    h = _ln_in(v, g[...], bb[...])
    h_ref[...] = h
    pq = _dot(h, wpq[...])
    p_ref[...] = pq[:, :_L]
    q_ref[...] = pq[:, _L:]


def _call_node_enc(x, w1, b1, w2, b2, w3, b3, g, bb, wpq, n):
    grid = (n // _BN,)
    row = pl.BlockSpec((_BN, _L), lambda i: (i, 0))
    return pl.pallas_call(
        _node_enc_body,
        grid=grid,
        in_specs=[row, _wspec(), _bspec(), _wspec(), _bspec(), _wspec(),
                  _bspec(), _bspec(), _bspec(),
                  pl.BlockSpec((_L, 2 * _L), lambda i: (0, 0))],
        out_specs=[row, row, row],
        out_shape=[jax.ShapeDtypeStruct((n, _L), jnp.float32)] * 3,
    )(x, w1, b1, w2, b2, w3, b3, g, bb, wpq)


def _edge_enc_body(rr_ref, w1p, w1d, b1, w2, b2, w3, b3, g, bb, he_ref):
    u = rr_ref[...] * (1.0 / RADIUS)
    dist = jnp.sqrt(jnp.sum(u * u, axis=-1, keepdims=True))
    t = jnp.maximum(_dot(u, w1p[...]) + dist * w1d[...] + b1[...], 0.0)
    t = jnp.maximum(_dot(t, w2[...]) + b2[...], 0.0)
    v = _dot(t, w3[...]) + b3[...]
    he_ref[...] = _ln_in(v, g[...], bb[...])


def _call_edge_enc(rr, w1p, w1d, b1, w2, b2, w3, b3, g, bb, e):
    grid = (e // _BE,)
    row16 = pl.BlockSpec((_BE, 16), lambda i: (i, 0))
    row = pl.BlockSpec((_BE, _L), lambda i: (i, 0))
    return pl.pallas_call(
        _edge_enc_body,
        grid=grid,
        in_specs=[row16, pl.BlockSpec((16, _L), lambda i: (0, 0)), _bspec(),
                  _bspec(), _wspec(), _bspec(), _wspec(), _bspec(),
                  _bspec(), _bspec()],
        out_specs=row,
        out_shape=jax.ShapeDtypeStruct((e, _L), jnp.float32),
    )(rr, w1p, w1d, b1, w2, b2, w3, b3, g, bb)


def _edge_step_body(g_ref, he_ref, w1, b1, w2, b2, w3, b3, g, bb,
                    heo_ref, hen_ref):
    he = he_ref[...]
    t = jnp.maximum(g_ref[...] + _dot(he, w1[...]) + b1[...], 0.0)
    t = jnp.maximum(_dot(t, w2[...]) + b2[...], 0.0)
    v = _dot(t, w3[...]) + b3[...]
    hn = _ln_in(v, g[...], bb[...])
    hen_ref[...] = hn
    heo_ref[...] = he + hn


def _call_edge_step(gg, he, w1, b1, w2, b2, w3, b3, g, bb, e):
    grid = (e // _BE,)
    row = pl.BlockSpec((_BE, _L), lambda i: (i, 0))
    return pl.pallas_call(
        _edge_step_body,
        grid=grid,
        in_specs=[row, row, _wspec(), _bspec(), _wspec(), _bspec(),
                  _wspec(), _bspec(), _bspec(), _bspec()],
        out_specs=[row, row],
        out_shape=[jax.ShapeDtypeStruct((e, _L), jnp.float32)] * 2,
    )(gg, he, w1, b1, w2, b2, w3, b3, g, bb)


def _node_step_body(h_ref, aggA_ref, aggB_ref, wh, wa, b1, w2, b2, w3, b3,
                    g, bb, wpq, ho_ref, p_ref, q_ref):
    h = h_ref[...]
    agg = aggA_ref[...] + aggB_ref[...]
    t = jnp.maximum(_dot(h, wh[...]) + _dot(agg, wa[...]) + b1[...], 0.0)
    t = jnp.maximum(_dot(t, w2[...]) + b2[...], 0.0)
    v = _dot(t, w3[...]) + b3[...]
    ho = h + _ln_in(v, g[...], bb[...])
    ho_ref[...] = ho
    pq = _dot(ho, wpq[...])
    p_ref[...] = pq[:, :_L]
    q_ref[...] = pq[:, _L:]


def _call_node_step(h, agg2, wh, wa, b1, w2, b2, w3, b3, g, bb, wpq, n):
    grid = (n // _BN,)
    row = pl.BlockSpec((_BN, _L), lambda i: (i, 0))
    rowB = pl.BlockSpec((_BN, _L), lambda i: (i + n // _BN, 0))
    return pl.pallas_call(
        _node_step_body,
        grid=grid,
        in_specs=[row, row, rowB, _wspec(), _wspec(), _bspec(), _wspec(),
                  _bspec(), _wspec(), _bspec(), _bspec(), _bspec(),
                  pl.BlockSpec((_L, 2 * _L), lambda i: (0, 0))],
        out_specs=[row, row, row],
        out_shape=[jax.ShapeDtypeStruct((n, _L), jnp.float32)] * 3,
    )(h, agg2, agg2, wh, wa, b1, w2, b2, w3, b3, g, bb, wpq)


def _node_last_body(h_ref, aggA_ref, aggB_ref, wh, wa, b1, w2, b2, w3, b3,
                    g, bb, ho_ref):
    h = h_ref[...]
    agg = aggA_ref[...] + aggB_ref[...]
    t = jnp.maximum(_dot(h, wh[...]) + _dot(agg, wa[...]) + b1[...], 0.0)
    t = jnp.maximum(_dot(t, w2[...]) + b2[...], 0.0)
    v = _dot(t, w3[...]) + b3[...]
    ho_ref[...] = h + _ln_in(v, g[...], bb[...])


def _call_node_last(h, agg2, wh, wa, b1, w2, b2, w3, b3, g, bb, n):
    grid = (n // _BN,)
    row = pl.BlockSpec((_BN, _L), lambda i: (i, 0))
    rowB = pl.BlockSpec((_BN, _L), lambda i: (i + n // _BN, 0))
    return pl.pallas_call(
        _node_last_body,
        grid=grid,
        in_specs=[row, row, rowB, _wspec(), _wspec(), _bspec(), _wspec(),
                  _bspec(), _wspec(), _bspec(), _bspec(), _bspec()],
        out_specs=row,
        out_shape=jax.ShapeDtypeStruct((n, _L), jnp.float32),
    )(h, agg2, agg2, wh, wa, b1, w2, b2, w3, b3, g, bb)


def _dec_body(h_ref, w1, b1, w2, b2, w3p, b3p, acc_ref):
    t = jnp.maximum(_dot(h_ref[...], w1[...]) + b1[...], 0.0)
    t = jnp.maximum(_dot(t, w2[...]) + b2[...], 0.0)
    acc_ref[...] = _dot(t, w3p[...]) + b3p[...]


def _call_dec(h, w1, b1, w2, b2, w3p, b3p, n):
    grid = (n // _BN,)
    row = pl.BlockSpec((_BN, _L), lambda i: (i, 0))
    return pl.pallas_call(
        _dec_body,
        grid=grid,
        in_specs=[row, _wspec(), _bspec(), _wspec(), _bspec(), _wspec(),
                  _bspec()],
        out_specs=row,
        out_shape=jax.ShapeDtypeStruct((n, _L), jnp.float32),
    )(h, w1, b1, w2, b2, w3p, b3p)


def _b(p):
    return p["b"].reshape(1, -1)


def kernel(position_sequence, particle_types, edge_index, n_particles_per_example, params):
    n = position_sequence.shape[0]
    e = edge_index.shape[1]
    boundaries = jnp.asarray(_BOUNDS)
    most_recent = position_sequence[:, -1]
    vel = position_sequence[:, 1:] - position_sequence[:, :-1]
    flat_vel = vel.reshape(n, -1)
    d_lo = most_recent - boundaries[:, 0][None]
    d_hi = boundaries[:, 1][None] - most_recent
    d_b = jnp.clip(jnp.concatenate([d_lo, d_hi], axis=1) / RADIUS, -1.0, 1.0)
    onehot = jax.nn.one_hot(particle_types, 9, dtype=jnp.float32)
    x = jnp.pad(jnp.concatenate([flat_vel, d_b, onehot], axis=1),
                ((0, 0), (0, _L - 30)))
    senders = edge_index[0]
    receivers = edge_index[1]

    prm = params
    steps = prm["steps"]

    ne = prm["node_enc"]
    w1n = ne[0]["W"]
    w1eff = jnp.concatenate([w1n[:21], prm["type_emb"] @ w1n[21:37]], axis=0)
    w1eff = jnp.pad(w1eff, ((0, _L - 30), (0, 0)))

    def _split_edge_w1(i):
        w = steps[i]["edge_mlp"][0]["W"]
        return w[:_L], w[_L:2 * _L], w[2 * _L:]

    ee = prm["edge_enc"]
    w1e = ee[0]["W"]
    w1p = jnp.pad(w1e[:3], ((0, 13), (0, 0)))
    w1d = w1e[3].reshape(1, -1)

    tpos = jnp.pad(most_recent, ((0, 0), (0, 13)))
    rr = tpos[senders] - tpos[receivers]
    he = _call_edge_enc(rr, w1p, w1d, _b(ee[0]), ee[1]["W"], _b(ee[1]),
                        ee[2]["W"], _b(ee[2]),
                        prm["edge_enc_ln"]["g"].reshape(1, -1),
                        prm["edge_enc_ln"]["b"].reshape(1, -1), e)

    w1s0, w1r0, _ = _split_edge_w1(0)
    wpq0 = jnp.concatenate([w1s0, w1r0], axis=1)
    h, p_tab, q_tab = _call_node_enc(
        x, w1eff, _b(ne[0]), ne[1]["W"], _b(ne[1]), ne[2]["W"], _b(ne[2]),
        prm["node_enc_ln"]["g"].reshape(1, -1),
        prm["node_enc_ln"]["b"].reshape(1, -1), wpq0, n)

    zeros_n = jnp.zeros(((-(-n // _NS) + 7) // 8 * 8, _L), jnp.float32)
    gather128 = _pair_gather(e, _L, +1)
    scatter = _scatter_add(n, e, _L)

    for i in range(len(steps)):
        sp = steps[i]
        em = sp["edge_mlp"]
        nm = sp["node_mlp"]
        _, _, w1c = _split_edge_w1(i)
        gg = gather128(p_tab, q_tab, senders, receivers)
        he, he_new = _call_edge_step(
            gg, he, w1c, _b(em[0]), em[1]["W"], _b(em[1]), em[2]["W"],
            _b(em[2]), sp["edge_ln"]["g"].reshape(1, -1),
            sp["edge_ln"]["b"].reshape(1, -1), e)
        agg2 = scatter(he_new, receivers, zeros_n)
        wn1 = nm[0]["W"]
        wh, wa = wn1[:_L], wn1[_L:]
        lng = sp["node_ln"]["g"].reshape(1, -1)
        lnb = sp["node_ln"]["b"].reshape(1, -1)
        if i + 1 < len(steps):
            w1s, w1r, _ = _split_edge_w1(i + 1)
            wpq = jnp.concatenate([w1s, w1r], axis=1)
            h, p_tab, q_tab = _call_node_step(
                h, agg2, wh, wa, _b(nm[0]), nm[1]["W"], _b(nm[1]),
                nm[2]["W"], _b(nm[2]), lng, lnb, wpq, n)
        else:
            h = _call_node_last(
                h, agg2, wh, wa, _b(nm[0]), nm[1]["W"], _b(nm[1]),
                nm[2]["W"], _b(nm[2]), lng, lnb, n)

    dec = prm["decoder"]
    w3p = jnp.pad(dec[2]["W"], ((0, 0), (0, _L - 3)))
    b3p = jnp.pad(dec[2]["b"], (0, _L - 3)).reshape(1, -1)
    acc = _call_dec(h, dec[0]["W"], _b(dec[0]), dec[1]["W"], _b(dec[1]),
                    w3p, b3p, n)[:, :3]

    prev_vel = position_sequence[:, -1] - position_sequence[:, -2]
    return position_sequence[:, -1] + prev_vel + acc


# speedup vs baseline: 3.3828x; 1.1939x over previous
import functools

import jax
import jax.numpy as jnp
import numpy as np
from jax import lax
from jax.experimental import pallas as pl
from jax.experimental.pallas import tpu as pltpu
from jax.experimental.pallas import tpu_sc as plsc

RADIUS = 0.015
_BOUNDS = np.array([[0.1, 0.9], [0.1, 0.9], [0.1, 0.9]], dtype=np.float32)
_EPS = 1e-5
_NC, _NS = 2, 16
_NW = _NC * _NS
_C = 40
_BN = 1000
_BE = 2560
_L = 128


def _pair_gather(n_edges, d, sign, c=80):
    ew = n_edges // _NW
    nch = ew // c
    assert ew * _NW == n_edges and nch * c == ew and nch >= 2
    assert c % 8 == 0 and c <= 128
    mesh = plsc.VectorSubcoreMesh(core_axis_name="c", subcore_axis_name="s")

    def body(a_hbm, b_hbm, s_hbm, r_hbm, out_hbm,
             si0, ri0, si1, ri1, bp0, bq0, bp1, bq1, sa0, sb0, sa1, sb1):
        wid = lax.axis_index("s") * _NC + lax.axis_index("c")
        base = pl.multiple_of(wid * ew, 8)

        def load_fire(j, si, ri, bp, bq, sa, sb):
            off = pl.multiple_of(base + j * c, 8)
            pltpu.sync_copy(s_hbm.at[pl.ds(off, c)], si)
            pltpu.sync_copy(r_hbm.at[pl.ds(off, c)], ri)
            pltpu.async_copy(a_hbm.at[si], bp, sa)
            pltpu.async_copy(b_hbm.at[ri], bq, sb)

        def wait(si, ri, bp, bq, sa, sb):
            pltpu.make_async_copy(a_hbm.at[si], bp, sa).wait()
            pltpu.make_async_copy(b_hbm.at[ri], bq, sb).wait()

        def combine_store(j, bp, bq):
            def row(i, cc):
                for g in range(d // 16):
                    sl = pl.ds(g * 16, 16)
                    if sign > 0:
                        bp[i, sl] = bp[i, sl] + bq[i, sl]
                    else:
                        bp[i, sl] = bp[i, sl] - bq[i, sl]
                return cc
            lax.fori_loop(0, c, row, 0)
            off = pl.multiple_of(base + j * c, 8)
            pltpu.sync_copy(bp, out_hbm.at[pl.ds(off, c)])

        load_fire(0, si0, ri0, bp0, bq0, sa0, sb0)
        load_fire(1, si1, ri1, bp1, bq1, sa1, sb1)

        def pair(p, cc):
            j0 = 2 * p
            wait(si0, ri0, bp0, bq0, sa0, sb0)
            combine_store(j0, bp0, bq0)

            @pl.when(j0 + 2 < nch)
            def _():
                load_fire(j0 + 2, si0, ri0, bp0, bq0, sa0, sb0)

            wait(si1, ri1, bp1, bq1, sa1, sb1)
            combine_store(j0 + 1, bp1, bq1)

            @pl.when(j0 + 3 < nch)
            def _():
                load_fire(j0 + 3, si1, ri1, bp1, bq1, sa1, sb1)
            return cc

        lax.fori_loop(0, nch // 2, pair, 0)
        if nch % 2 == 1:
            wait(si0, ri0, bp0, bq0, sa0, sb0)
            combine_store(nch - 1, bp0, bq0)

    return functools.partial(
        pl.kernel, body,
        out_type=jax.ShapeDtypeStruct((n_edges, d), jnp.float32),
        mesh=mesh,
        compiler_params=pltpu.CompilerParams(use_tc_tiling_on_sc=(d % 128 == 0)),
        scratch_types=[
            pltpu.VMEM((c,), jnp.int32), pltpu.VMEM((c,), jnp.int32),
            pltpu.VMEM((c,), jnp.int32), pltpu.VMEM((c,), jnp.int32),
            pltpu.VMEM((c, d), jnp.float32), pltpu.VMEM((c, d), jnp.float32),
            pltpu.VMEM((c, d), jnp.float32), pltpu.VMEM((c, d), jnp.float32),
            pltpu.SemaphoreType.DMA, pltpu.SemaphoreType.DMA,
            pltpu.SemaphoreType.DMA, pltpu.SemaphoreType.DMA,
        ],
    )()


def _scatter_add(n_nodes, n_edges, d):
    c = 80
    ew = n_edges // _NW
    nch = ew // c
    rt = (-(-n_nodes // _NS) + 7) // 8 * 8
    n_pad = rt * _NS
    last = n_nodes - rt * (_NS - 1)
    assert nch * c == ew and nch >= 2
    assert last > 0 and last % 8 == 0 and n_nodes % 8 == 0
    mesh = plsc.VectorSubcoreMesh(core_axis_name="c", subcore_axis_name="s")

    def body(v_hbm, r_hbm, z_hbm, out_hbm, ri0, ri1, b0, b1, sa0, sa1, shared):
        cid = lax.axis_index("c")
        sid = lax.axis_index("s")
        wid = sid * _NC + cid
        base = pl.multiple_of(wid * ew, 8)
        roff = pl.multiple_of(sid * rt, 8)

        pltpu.sync_copy(z_hbm, shared.at[pl.ds(roff, rt)])
        plsc.subcore_barrier()

        def load(j, ri, b, sa):
            off = pl.multiple_of(base + j * c, 8)
            pltpu.sync_copy(r_hbm.at[pl.ds(off, c)], ri)
            pltpu.async_copy(v_hbm.at[pl.ds(off, c)], b, sa)

        def wait(b, sa):
            pltpu.make_async_copy(v_hbm.at[pl.ds(0, c)], b, sa).wait()

        load(0, ri0, b0, sa0)
        load(1, ri1, b1, sa1)

        def pair(p, cc):
            j0 = 2 * p
            wait(b0, sa0)
            pltpu.sync_copy(b0, shared.at[ri0], add=True)

            @pl.when(j0 + 2 < nch)
            def _():
                load(j0 + 2, ri0, b0, sa0)

            wait(b1, sa1)
            pltpu.sync_copy(b1, shared.at[ri1], add=True)

            @pl.when(j0 + 3 < nch)
            def _():
                load(j0 + 3, ri1, b1, sa1)
            return cc

        lax.fori_loop(0, nch // 2, pair, 0)
        if nch % 2 == 1:
            wait(b0, sa0)
            pltpu.sync_copy(b0, shared.at[ri0], add=True)

        plsc.subcore_barrier()
        obase = pl.multiple_of(cid * n_nodes + roff, 8)

        @pl.when(sid == _NS - 1)
        def _dump_last():
            pltpu.sync_copy(shared.at[pl.ds(roff, last)],
                            out_hbm.at[pl.ds(obase, last)])

        @pl.when(sid < _NS - 1)
        def _dump_full():
            pltpu.sync_copy(shared.at[pl.ds(roff, rt)],
                            out_hbm.at[pl.ds(obase, rt)])

    return functools.partial(
        pl.kernel, body,
        out_type=jax.ShapeDtypeStruct((2 * n_nodes, d), jnp.float32),
        mesh=mesh,
        scratch_types=[
            pltpu.VMEM((c,), jnp.int32), pltpu.VMEM((c,), jnp.int32),
            pltpu.VMEM((c, d), jnp.float32), pltpu.VMEM((c, d), jnp.float32),
            pltpu.SemaphoreType.DMA, pltpu.SemaphoreType.DMA,
            pltpu.VMEM_SHARED((n_pad, d), jnp.float32),
        ],
    )()


def _ln_in(v, g, b):
    m = jnp.mean(v, axis=-1, keepdims=True)
    var = jnp.mean((v - m) ** 2, axis=-1, keepdims=True)
    return (v - m) / jnp.sqrt(var + _EPS) * g + b


def _dot(a, b):
    return jnp.dot(a, b, preferred_element_type=jnp.float32)


def _wspec():
    return pl.BlockSpec((_L, _L), lambda i: (0, 0))


def _bspec():
    return pl.BlockSpec((1, _L), lambda i: (0, 0))


def _node_enc_body(x_ref, w1, b1, w2, b2, w3, b3, g, bb, wpq,
                   h_ref, p_ref, q_ref):
    t = jnp.maximum(_dot(x_ref[...], w1[...]) + b1[...], 0.0)
    t = jnp.maximum(_dot(t, w2[...]) + b2[...], 0.0)
    v = _dot(t, w3[...]) + b3[...]
    h = _ln_in(v, g[...], bb[...])
    h_ref[...] = h
    pq = _dot(h, wpq[...])
    p_ref[...] = pq[:, :_L]
    q_ref[...] = pq[:, _L:]


def _call_node_enc(x, w1, b1, w2, b2, w3, b3, g, bb, wpq, n):
    grid = (n // _BN,)
    row = pl.BlockSpec((_BN, _L), lambda i: (i, 0))
    return pl.pallas_call(
        _node_enc_body,
        grid=grid,
        in_specs=[row, _wspec(), _bspec(), _wspec(), _bspec(), _wspec(),
                  _bspec(), _bspec(), _bspec(),
                  pl.BlockSpec((_L, 2 * _L), lambda i: (0, 0))],
        out_specs=[row, row, row],
        out_shape=[jax.ShapeDtypeStruct((n, _L), jnp.float32)] * 3,
    )(x, w1, b1, w2, b2, w3, b3, g, bb, wpq)


def _edge_enc_body(rr_ref, w1p, w1d, b1, w2, b2, w3, b3, g, bb, he_ref):
    u = rr_ref[...] * (1.0 / RADIUS)
    dist = jnp.sqrt(jnp.sum(u * u, axis=-1, keepdims=True))
    t = jnp.maximum(_dot(u, w1p[...]) + dist * w1d[...] + b1[...], 0.0)
    t = jnp.maximum(_dot(t, w2[...]) + b2[...], 0.0)
    v = _dot(t, w3[...]) + b3[...]
    he_ref[...] = _ln_in(v, g[...], bb[...])


def _call_edge_enc(rr, w1p, w1d, b1, w2, b2, w3, b3, g, bb, e):
    grid = (e // _BE,)
    row16 = pl.BlockSpec((_BE, 16), lambda i: (i, 0))
    row = pl.BlockSpec((_BE, _L), lambda i: (i, 0))
    return pl.pallas_call(
        _edge_enc_body,
        grid=grid,
        in_specs=[row16, pl.BlockSpec((16, _L), lambda i: (0, 0)), _bspec(),
                  _bspec(), _wspec(), _bspec(), _wspec(), _bspec(),
                  _bspec(), _bspec()],
        out_specs=row,
        out_shape=jax.ShapeDtypeStruct((e, _L), jnp.float32),
    )(rr, w1p, w1d, b1, w2, b2, w3, b3, g, bb)


def _edge_step_body(g_ref, he_ref, w1, b1, w2, b2, w3, b3, g, bb,
                    heo_ref, hen_ref):
    he = he_ref[...]
    t = jnp.maximum(g_ref[...] + _dot(he, w1[...]) + b1[...], 0.0)
    t = jnp.maximum(_dot(t, w2[...]) + b2[...], 0.0)
    v = _dot(t, w3[...]) + b3[...]
    hn = _ln_in(v, g[...], bb[...])
    hen_ref[...] = hn
    heo_ref[...] = he + hn


def _call_edge_step(gg, he, w1, b1, w2, b2, w3, b3, g, bb, e):
    grid = (e // _BE,)
    row = pl.BlockSpec((_BE, _L), lambda i: (i, 0))
    return pl.pallas_call(
        _edge_step_body,
        grid=grid,
        in_specs=[row, row, _wspec(), _bspec(), _wspec(), _bspec(),
                  _wspec(), _bspec(), _bspec(), _bspec()],
        out_specs=[row, row],
        out_shape=[jax.ShapeDtypeStruct((e, _L), jnp.float32)] * 2,
    )(gg, he, w1, b1, w2, b2, w3, b3, g, bb)


def _node_step_body(h_ref, aggA_ref, aggB_ref, wh, wa, b1, w2, b2, w3, b3,
                    g, bb, wpq, ho_ref, p_ref, q_ref):
    h = h_ref[...]
    agg = aggA_ref[...] + aggB_ref[...]
    t = jnp.maximum(_dot(h, wh[...]) + _dot(agg, wa[...]) + b1[...], 0.0)
    t = jnp.maximum(_dot(t, w2[...]) + b2[...], 0.0)
    v = _dot(t, w3[...]) + b3[...]
    ho = h + _ln_in(v, g[...], bb[...])
    ho_ref[...] = ho
    pq = _dot(ho, wpq[...])
    p_ref[...] = pq[:, :_L]
    q_ref[...] = pq[:, _L:]


def _call_node_step(h, agg2, wh, wa, b1, w2, b2, w3, b3, g, bb, wpq, n):
    grid = (n // _BN,)
    row = pl.BlockSpec((_BN, _L), lambda i: (i, 0))
    rowB = pl.BlockSpec((_BN, _L), lambda i: (i + n // _BN, 0))
    return pl.pallas_call(
        _node_step_body,
        grid=grid,
        in_specs=[row, row, rowB, _wspec(), _wspec(), _bspec(), _wspec(),
                  _bspec(), _wspec(), _bspec(), _bspec(), _bspec(),
                  pl.BlockSpec((_L, 2 * _L), lambda i: (0, 0))],
        out_specs=[row, row, row],
        out_shape=[jax.ShapeDtypeStruct((n, _L), jnp.float32)] * 3,
    )(h, agg2, agg2, wh, wa, b1, w2, b2, w3, b3, g, bb, wpq)


def _node_last_body(h_ref, aggA_ref, aggB_ref, wh, wa, b1, w2, b2, w3, b3,
                    g, bb, ho_ref):
    h = h_ref[...]
    agg = aggA_ref[...] + aggB_ref[...]
    t = jnp.maximum(_dot(h, wh[...]) + _dot(agg, wa[...]) + b1[...], 0.0)
    t = jnp.maximum(_dot(t, w2[...]) + b2[...], 0.0)
    v = _dot(t, w3[...]) + b3[...]
    ho_ref[...] = h + _ln_in(v, g[...], bb[...])


def _call_node_last(h, agg2, wh, wa, b1, w2, b2, w3, b3, g, bb, n):
    grid = (n // _BN,)
    row = pl.BlockSpec((_BN, _L), lambda i: (i, 0))
    rowB = pl.BlockSpec((_BN, _L), lambda i: (i + n // _BN, 0))
    return pl.pallas_call(
        _node_last_body,
        grid=grid,
        in_specs=[row, row, rowB, _wspec(), _wspec(), _bspec(), _wspec(),
                  _bspec(), _wspec(), _bspec(), _bspec(), _bspec()],
        out_specs=row,
        out_shape=jax.ShapeDtypeStruct((n, _L), jnp.float32),
    )(h, agg2, agg2, wh, wa, b1, w2, b2, w3, b3, g, bb)


def _dec_body(h_ref, w1, b1, w2, b2, w3p, b3p, acc_ref):
    t = jnp.maximum(_dot(h_ref[...], w1[...]) + b1[...], 0.0)
    t = jnp.maximum(_dot(t, w2[...]) + b2[...], 0.0)
    acc_ref[...] = _dot(t, w3p[...]) + b3p[...]


def _call_dec(h, w1, b1, w2, b2, w3p, b3p, n):
    grid = (n // _BN,)
    row = pl.BlockSpec((_BN, _L), lambda i: (i, 0))
    return pl.pallas_call(
        _dec_body,
        grid=grid,
        in_specs=[row, _wspec(), _bspec(), _wspec(), _bspec(), _wspec(),
                  _bspec()],
        out_specs=row,
        out_shape=jax.ShapeDtypeStruct((n, _L), jnp.float32),
    )(h, w1, b1, w2, b2, w3p, b3p)


def _b(p):
    return p["b"].reshape(1, -1)


def kernel(position_sequence, particle_types, edge_index, n_particles_per_example, params):
    n = position_sequence.shape[0]
    e = edge_index.shape[1]
    boundaries = jnp.asarray(_BOUNDS)
    most_recent = position_sequence[:, -1]
    vel = position_sequence[:, 1:] - position_sequence[:, :-1]
    flat_vel = vel.reshape(n, -1)
    d_lo = most_recent - boundaries[:, 0][None]
    d_hi = boundaries[:, 1][None] - most_recent
    d_b = jnp.clip(jnp.concatenate([d_lo, d_hi], axis=1) / RADIUS, -1.0, 1.0)
    onehot = jax.nn.one_hot(particle_types, 9, dtype=jnp.float32)
    x = jnp.pad(jnp.concatenate([flat_vel, d_b, onehot], axis=1),
                ((0, 0), (0, _L - 30)))
    senders = edge_index[0]
    receivers = edge_index[1]

    prm = params
    steps = prm["steps"]

    ne = prm["node_enc"]
    w1n = ne[0]["W"]
    w1eff = jnp.concatenate([w1n[:21], prm["type_emb"] @ w1n[21:37]], axis=0)
    w1eff = jnp.pad(w1eff, ((0, _L - 30), (0, 0)))

    def _split_edge_w1(i):
        w = steps[i]["edge_mlp"][0]["W"]
        return w[:_L], w[_L:2 * _L], w[2 * _L:]

    ee = prm["edge_enc"]
    w1e = ee[0]["W"]
    w1p = jnp.pad(w1e[:3], ((0, 13), (0, 0)))
    w1d = w1e[3].reshape(1, -1)

    tpos = jnp.pad(most_recent, ((0, 0), (0, 13)))
    rr = tpos[senders] - tpos[receivers]
    he = _call_edge_enc(rr, w1p, w1d, _b(ee[0]), ee[1]["W"], _b(ee[1]),
                        ee[2]["W"], _b(ee[2]),
                        prm["edge_enc_ln"]["g"].reshape(1, -1),
                        prm["edge_enc_ln"]["b"].reshape(1, -1), e)

    w1s0, w1r0, _ = _split_edge_w1(0)
    wpq0 = jnp.concatenate([w1s0, w1r0], axis=1)
    h, p_tab, q_tab = _call_node_enc(
        x, w1eff, _b(ne[0]), ne[1]["W"], _b(ne[1]), ne[2]["W"], _b(ne[2]),
        prm["node_enc_ln"]["g"].reshape(1, -1),
        prm["node_enc_ln"]["b"].reshape(1, -1), wpq0, n)

    zeros_n = jnp.zeros(((-(-n // _NS) + 7) // 8 * 8, _L), jnp.float32)
    gather128 = _pair_gather(e, _L, +1)
    scatter = _scatter_add(n, e, _L)

    for i in range(len(steps)):
        sp = steps[i]
        em = sp["edge_mlp"]
        nm = sp["node_mlp"]
        _, _, w1c = _split_edge_w1(i)
        gg = gather128(p_tab, q_tab, senders, receivers)
        he, he_new = _call_edge_step(
            gg, he, w1c, _b(em[0]), em[1]["W"], _b(em[1]), em[2]["W"],
            _b(em[2]), sp["edge_ln"]["g"].reshape(1, -1),
            sp["edge_ln"]["b"].reshape(1, -1), e)
        agg2 = scatter(he_new, receivers, zeros_n)
        wn1 = nm[0]["W"]
        wh, wa = wn1[:_L], wn1[_L:]
        lng = sp["node_ln"]["g"].reshape(1, -1)
        lnb = sp["node_ln"]["b"].reshape(1, -1)
        if i + 1 < len(steps):
            w1s, w1r, _ = _split_edge_w1(i + 1)
            wpq = jnp.concatenate([w1s, w1r], axis=1)
            h, p_tab, q_tab = _call_node_step(
                h, agg2, wh, wa, _b(nm[0]), nm[1]["W"], _b(nm[1]),
                nm[2]["W"], _b(nm[2]), lng, lnb, wpq, n)
        else:
            h = _call_node_last(
                h, agg2, wh, wa, _b(nm[0]), nm[1]["W"], _b(nm[1]),
                nm[2]["W"], _b(nm[2]), lng, lnb, n)

    dec = prm["decoder"]
    w3p = jnp.pad(dec[2]["W"], ((0, 0), (0, _L - 3)))
    b3p = jnp.pad(dec[2]["b"], (0, _L - 3)).reshape(1, -1)
    acc = _call_dec(h, dec[0]["W"], _b(dec[0]), dec[1]["W"], _b(dec[1]),
                    w3p, b3p, n)[:, :3]

    prev_vel = position_sequence[:, -1] - position_sequence[:, -2]
    return position_sequence[:, -1] + prev_vel + acc


# speedup vs baseline: 3.6625x; 1.0827x over previous
import functools

import jax
import jax.numpy as jnp
import numpy as np
from jax import lax
from jax.experimental import pallas as pl
from jax.experimental.pallas import tpu as pltpu
from jax.experimental.pallas import tpu_sc as plsc

RADIUS = 0.015
_BOUNDS = np.array([[0.1, 0.9], [0.1, 0.9], [0.1, 0.9]], dtype=np.float32)
_EPS = 1e-5
_NC, _NS = 2, 16
_NW = _NC * _NS
_C = 40
_BN = 1000
_BE = 2560
_L = 128


def _pair_gather(n_edges, d, sign, c=80):
    ew = n_edges // _NW
    nch = ew // c
    assert ew * _NW == n_edges and nch * c == ew and nch >= 2
    assert c % 8 == 0 and c <= 128
    mesh = plsc.VectorSubcoreMesh(core_axis_name="c", subcore_axis_name="s")

    def body(a_hbm, b_hbm, s_hbm, r_hbm, out_hbm,
             si0, ri0, si1, ri1, bp0, bq0, bp1, bq1, sa0, sb0, sa1, sb1):
        wid = lax.axis_index("s") * _NC + lax.axis_index("c")
        base = pl.multiple_of(wid * ew, 8)

        def load_fire(j, si, ri, bp, bq, sa, sb):
            off = pl.multiple_of(base + j * c, 8)
            pltpu.sync_copy(s_hbm.at[pl.ds(off, c)], si)
            pltpu.sync_copy(r_hbm.at[pl.ds(off, c)], ri)
            pltpu.async_copy(a_hbm.at[si], bp, sa)
            pltpu.async_copy(b_hbm.at[ri], bq, sb)

        def wait(si, ri, bp, bq, sa, sb):
            pltpu.make_async_copy(a_hbm.at[si], bp, sa).wait()
            pltpu.make_async_copy(b_hbm.at[ri], bq, sb).wait()

        def combine_store(j, bp, bq):
            def row(i, cc):
                for g in range(d // 16):
                    sl = pl.ds(g * 16, 16)
                    if sign > 0:
                        bp[i, sl] = bp[i, sl] + bq[i, sl]
                    else:
                        bp[i, sl] = bp[i, sl] - bq[i, sl]
                return cc
            lax.fori_loop(0, c, row, 0)
            off = pl.multiple_of(base + j * c, 8)
            pltpu.sync_copy(bp, out_hbm.at[pl.ds(off, c)])

        load_fire(0, si0, ri0, bp0, bq0, sa0, sb0)
        load_fire(1, si1, ri1, bp1, bq1, sa1, sb1)

        def pair(p, cc):
            j0 = 2 * p
            wait(si0, ri0, bp0, bq0, sa0, sb0)
            combine_store(j0, bp0, bq0)

            @pl.when(j0 + 2 < nch)
            def _():
                load_fire(j0 + 2, si0, ri0, bp0, bq0, sa0, sb0)

            wait(si1, ri1, bp1, bq1, sa1, sb1)
            combine_store(j0 + 1, bp1, bq1)

            @pl.when(j0 + 3 < nch)
            def _():
                load_fire(j0 + 3, si1, ri1, bp1, bq1, sa1, sb1)
            return cc

        lax.fori_loop(0, nch // 2, pair, 0)
        if nch % 2 == 1:
            wait(si0, ri0, bp0, bq0, sa0, sb0)
            combine_store(nch - 1, bp0, bq0)

    return functools.partial(
        pl.kernel, body,
        out_type=jax.ShapeDtypeStruct((n_edges, d), jnp.float32),
        mesh=mesh,
        compiler_params=pltpu.CompilerParams(use_tc_tiling_on_sc=(d % 128 == 0)),
        scratch_types=[
            pltpu.VMEM((c,), jnp.int32), pltpu.VMEM((c,), jnp.int32),
            pltpu.VMEM((c,), jnp.int32), pltpu.VMEM((c,), jnp.int32),
            pltpu.VMEM((c, d), jnp.float32), pltpu.VMEM((c, d), jnp.float32),
            pltpu.VMEM((c, d), jnp.float32), pltpu.VMEM((c, d), jnp.float32),
            pltpu.SemaphoreType.DMA, pltpu.SemaphoreType.DMA,
            pltpu.SemaphoreType.DMA, pltpu.SemaphoreType.DMA,
        ],
    )()


def _scatter_add(n_nodes, n_edges, d, c=80):
    ew = n_edges // _NW
    nch = ew // c
    rt = (-(-n_nodes // _NS) + 7) // 8 * 8
    n_pad = rt * _NS
    last = n_nodes - rt * (_NS - 1)
    assert nch * c == ew and nch >= 2
    assert last > 0 and last % 8 == 0 and n_nodes % 8 == 0
    mesh = plsc.VectorSubcoreMesh(core_axis_name="c", subcore_axis_name="s")

    def body(v_hbm, r_hbm, z_hbm, out_hbm, ri0, ri1, b0, b1, sa0, sa1, shared):
        cid = lax.axis_index("c")
        sid = lax.axis_index("s")
        wid = sid * _NC + cid
        base = pl.multiple_of(wid * ew, 8)
        roff = pl.multiple_of(sid * rt, 8)

        pltpu.sync_copy(z_hbm, shared.at[pl.ds(roff, rt)])
        plsc.subcore_barrier()

        def load(j, ri, b, sa):
            off = pl.multiple_of(base + j * c, 8)
            pltpu.sync_copy(r_hbm.at[pl.ds(off, c)], ri)
            pltpu.async_copy(v_hbm.at[pl.ds(off, c)], b, sa)

        def wait(b, sa):
            pltpu.make_async_copy(v_hbm.at[pl.ds(0, c)], b, sa).wait()

        load(0, ri0, b0, sa0)
        load(1, ri1, b1, sa1)

        def pair(p, cc):
            j0 = 2 * p
            wait(b0, sa0)
            pltpu.sync_copy(b0, shared.at[ri0], add=True)

            @pl.when(j0 + 2 < nch)
            def _():
                load(j0 + 2, ri0, b0, sa0)

            wait(b1, sa1)
            pltpu.sync_copy(b1, shared.at[ri1], add=True)

            @pl.when(j0 + 3 < nch)
            def _():
                load(j0 + 3, ri1, b1, sa1)
            return cc

        lax.fori_loop(0, nch // 2, pair, 0)
        if nch % 2 == 1:
            wait(b0, sa0)
            pltpu.sync_copy(b0, shared.at[ri0], add=True)

        plsc.subcore_barrier()
        obase = pl.multiple_of(cid * n_nodes + roff, 8)

        @pl.when(sid == _NS - 1)
        def _dump_last():
            pltpu.sync_copy(shared.at[pl.ds(roff, last)],
                            out_hbm.at[pl.ds(obase, last)])

        @pl.when(sid < _NS - 1)
        def _dump_full():
            pltpu.sync_copy(shared.at[pl.ds(roff, rt)],
                            out_hbm.at[pl.ds(obase, rt)])

    return functools.partial(
        pl.kernel, body,
        out_type=jax.ShapeDtypeStruct((2 * n_nodes, d), jnp.float32),
        mesh=mesh,
        scratch_types=[
            pltpu.VMEM((c,), jnp.int32), pltpu.VMEM((c,), jnp.int32),
            pltpu.VMEM((c, d), jnp.float32), pltpu.VMEM((c, d), jnp.float32),
            pltpu.SemaphoreType.DMA, pltpu.SemaphoreType.DMA,
            pltpu.VMEM_SHARED((n_pad, d), jnp.float32),
        ],
    )()


def _ln_in(v, g, b):
    m = jnp.mean(v, axis=-1, keepdims=True)
    var = jnp.mean((v - m) ** 2, axis=-1, keepdims=True)
    return (v - m) / jnp.sqrt(var + _EPS) * g + b


def _dot(a, b):
    return jnp.dot(a, b, preferred_element_type=jnp.float32)


def _wspec():
    return pl.BlockSpec((_L, _L), lambda i: (0, 0))


def _bspec():
    return pl.BlockSpec((1, _L), lambda i: (0, 0))


def _node_enc_body(x_ref, w1, b1, w2, b2, w3, b3, g, bb, wpq,
                   h_ref, p_ref, q_ref):
    t = jnp.maximum(_dot(x_ref[...], w1[...]) + b1[...], 0.0)
    t = jnp.maximum(_dot(t, w2[...]) + b2[...], 0.0)
    v = _dot(t, w3[...]) + b3[...]
    h = _ln_in(v, g[...], bb[...])
    h_ref[...] = h
    pq = _dot(h, wpq[...])
    p_ref[...] = pq[:, :_L]
    q_ref[...] = pq[:, _L:]


def _call_node_enc(x, w1, b1, w2, b2, w3, b3, g, bb, wpq, n):
    grid = (n // _BN,)
    row = pl.BlockSpec((_BN, _L), lambda i: (i, 0))
    return pl.pallas_call(
        _node_enc_body,
        grid=grid,
        in_specs=[row, _wspec(), _bspec(), _wspec(), _bspec(), _wspec(),
                  _bspec(), _bspec(), _bspec(),
                  pl.BlockSpec((_L, 2 * _L), lambda i: (0, 0))],
        out_specs=[row, row, row],
        out_shape=[jax.ShapeDtypeStruct((n, _L), jnp.float32)] * 3,
    )(x, w1, b1, w2, b2, w3, b3, g, bb, wpq)


def _edge_enc_body(rr_ref, w1p, w1d, b1, w2, b2, w3, b3, g, bb, he_ref):
    u = rr_ref[...] * (1.0 / RADIUS)
    dist = jnp.sqrt(jnp.sum(u * u, axis=-1, keepdims=True))
    t = jnp.maximum(_dot(u, w1p[...]) + dist * w1d[...] + b1[...], 0.0)
    t = jnp.maximum(_dot(t, w2[...]) + b2[...], 0.0)
    v = _dot(t, w3[...]) + b3[...]
    he_ref[...] = _ln_in(v, g[...], bb[...])


def _call_edge_enc(rr, w1p, w1d, b1, w2, b2, w3, b3, g, bb, e, be=_BE):
    grid = (e // be,)
    row16 = pl.BlockSpec((be, 16), lambda i: (i, 0))
    row = pl.BlockSpec((be, _L), lambda i: (i, 0))
    return pl.pallas_call(
        _edge_enc_body,
        grid=grid,
        in_specs=[row16, pl.BlockSpec((16, _L), lambda i: (0, 0)), _bspec(),
                  _bspec(), _wspec(), _bspec(), _wspec(), _bspec(),
                  _bspec(), _bspec()],
        out_specs=row,
        out_shape=jax.ShapeDtypeStruct((e, _L), jnp.float32),
    )(rr, w1p, w1d, b1, w2, b2, w3, b3, g, bb)


def _edge_step_body(g_ref, he_ref, w1, b1, w2, b2, w3, b3, g, bb,
                    heo_ref, hen_ref):
    he = he_ref[...]
    t = jnp.maximum(g_ref[...] + _dot(he, w1[...]) + b1[...], 0.0)
    t = jnp.maximum(_dot(t, w2[...]) + b2[...], 0.0)
    v = _dot(t, w3[...]) + b3[...]
    hn = _ln_in(v, g[...], bb[...])
    hen_ref[...] = hn
    heo_ref[...] = he + hn


def _call_edge_step(gg, he, w1, b1, w2, b2, w3, b3, g, bb, e, be=_BE):
    grid = (e // be,)
    row = pl.BlockSpec((be, _L), lambda i: (i, 0))
    return pl.pallas_call(
        _edge_step_body,
        grid=grid,
        in_specs=[row, row, _wspec(), _bspec(), _wspec(), _bspec(),
                  _wspec(), _bspec(), _bspec(), _bspec()],
        out_specs=[row, row],
        out_shape=[jax.ShapeDtypeStruct((e, _L), jnp.float32)] * 2,
    )(gg, he, w1, b1, w2, b2, w3, b3, g, bb)


def _node_step_body(h_ref, aggA_ref, aggB_ref, aggC_ref, aggD_ref,
                    wh, wa, b1, w2, b2, w3, b3,
                    g, bb, wpq, ho_ref, p_ref, q_ref):
    h = h_ref[...]
    agg = (aggA_ref[...] + aggB_ref[...]) + (aggC_ref[...] + aggD_ref[...])
    t = jnp.maximum(_dot(h, wh[...]) + _dot(agg, wa[...]) + b1[...], 0.0)
    t = jnp.maximum(_dot(t, w2[...]) + b2[...], 0.0)
    v = _dot(t, w3[...]) + b3[...]
    ho = h + _ln_in(v, g[...], bb[...])
    ho_ref[...] = ho
    pq = _dot(ho, wpq[...])
    p_ref[...] = pq[:, :_L]
    q_ref[...] = pq[:, _L:]


def _call_node_step(h, agg2a, agg2b, wh, wa, b1, w2, b2, w3, b3, g, bb, wpq, n):
    grid = (n // _BN,)
    row = pl.BlockSpec((_BN, _L), lambda i: (i, 0))
    rowB = pl.BlockSpec((_BN, _L), lambda i: (i + n // _BN, 0))
    return pl.pallas_call(
        _node_step_body,
        grid=grid,
        in_specs=[row, row, rowB, row, rowB, _wspec(), _wspec(), _bspec(),
                  _wspec(), _bspec(), _wspec(), _bspec(), _bspec(), _bspec(),
                  pl.BlockSpec((_L, 2 * _L), lambda i: (0, 0))],
        out_specs=[row, row, row],
        out_shape=[jax.ShapeDtypeStruct((n, _L), jnp.float32)] * 3,
    )(h, agg2a, agg2a, agg2b, agg2b, wh, wa, b1, w2, b2, w3, b3, g, bb, wpq)


def _node_last_body(h_ref, aggA_ref, aggB_ref, aggC_ref, aggD_ref,
                    wh, wa, b1, w2, b2, w3, b3, g, bb, ho_ref):
    h = h_ref[...]
    agg = (aggA_ref[...] + aggB_ref[...]) + (aggC_ref[...] + aggD_ref[...])
    t = jnp.maximum(_dot(h, wh[...]) + _dot(agg, wa[...]) + b1[...], 0.0)
    t = jnp.maximum(_dot(t, w2[...]) + b2[...], 0.0)
    v = _dot(t, w3[...]) + b3[...]
    ho_ref[...] = h + _ln_in(v, g[...], bb[...])


def _call_node_last(h, agg2a, agg2b, wh, wa, b1, w2, b2, w3, b3, g, bb, n):
    grid = (n // _BN,)
    row = pl.BlockSpec((_BN, _L), lambda i: (i, 0))
    rowB = pl.BlockSpec((_BN, _L), lambda i: (i + n // _BN, 0))
    return pl.pallas_call(
        _node_last_body,
        grid=grid,
        in_specs=[row, row, rowB, row, rowB, _wspec(), _wspec(), _bspec(),
                  _wspec(), _bspec(), _wspec(), _bspec(), _bspec(), _bspec()],
        out_specs=row,
        out_shape=jax.ShapeDtypeStruct((n, _L), jnp.float32),
    )(h, agg2a, agg2a, agg2b, agg2b, wh, wa, b1, w2, b2, w3, b3, g, bb)


def _dec_body(h_ref, w1, b1, w2, b2, w3p, b3p, acc_ref):
    t = jnp.maximum(_dot(h_ref[...], w1[...]) + b1[...], 0.0)
    t = jnp.maximum(_dot(t, w2[...]) + b2[...], 0.0)
    acc_ref[...] = _dot(t, w3p[...]) + b3p[...]


def _call_dec(h, w1, b1, w2, b2, w3p, b3p, n):
    grid = (n // _BN,)
    row = pl.BlockSpec((_BN, _L), lambda i: (i, 0))
    return pl.pallas_call(
        _dec_body,
        grid=grid,
        in_specs=[row, _wspec(), _bspec(), _wspec(), _bspec(), _wspec(),
                  _bspec()],
        out_specs=row,
        out_shape=jax.ShapeDtypeStruct((n, _L), jnp.float32),
    )(h, w1, b1, w2, b2, w3p, b3p)


def _b(p):
    return p["b"].reshape(1, -1)


def kernel(position_sequence, particle_types, edge_index, n_particles_per_example, params):
    n = position_sequence.shape[0]
    e = edge_index.shape[1]
    boundaries = jnp.asarray(_BOUNDS)
    most_recent = position_sequence[:, -1]
    vel = position_sequence[:, 1:] - position_sequence[:, :-1]
    flat_vel = vel.reshape(n, -1)
    d_lo = most_recent - boundaries[:, 0][None]
    d_hi = boundaries[:, 1][None] - most_recent
    d_b = jnp.clip(jnp.concatenate([d_lo, d_hi], axis=1) / RADIUS, -1.0, 1.0)
    onehot = jax.nn.one_hot(particle_types, 9, dtype=jnp.float32)
    x = jnp.pad(jnp.concatenate([flat_vel, d_b, onehot], axis=1),
                ((0, 0), (0, _L - 30)))
    senders = edge_index[0]
    receivers = edge_index[1]

    prm = params
    steps = prm["steps"]

    ne = prm["node_enc"]
    w1n = ne[0]["W"]
    w1eff = jnp.concatenate([w1n[:21], prm["type_emb"] @ w1n[21:37]], axis=0)
    w1eff = jnp.pad(w1eff, ((0, _L - 30), (0, 0)))

    def _split_edge_w1(i):
        w = steps[i]["edge_mlp"][0]["W"]
        return w[:_L], w[_L:2 * _L], w[2 * _L:]

    ee = prm["edge_enc"]
    w1e = ee[0]["W"]
    w1p = jnp.pad(w1e[:3], ((0, 13), (0, 0)))
    w1d = w1e[3].reshape(1, -1)

    e2 = e // 2
    be2 = 3200
    s_h = [senders[:e2], senders[e2:]]
    r_h = [receivers[:e2], receivers[e2:]]

    tpos = jnp.pad(most_recent, ((0, 0), (0, 13)))
    posg = _pair_gather(e2, 16, -1, c=40)
    enc_ln_g = prm["edge_enc_ln"]["g"].reshape(1, -1)
    enc_ln_b = prm["edge_enc_ln"]["b"].reshape(1, -1)
    he_h = []
    for hlf in range(2):
        rr = posg(tpos, tpos, s_h[hlf], r_h[hlf])
        he_h.append(_call_edge_enc(
            rr, w1p, w1d, _b(ee[0]), ee[1]["W"], _b(ee[1]), ee[2]["W"],
            _b(ee[2]), enc_ln_g, enc_ln_b, e2, be=be2))

    w1s0, w1r0, _ = _split_edge_w1(0)
    wpq0 = jnp.concatenate([w1s0, w1r0], axis=1)
    h, p_tab, q_tab = _call_node_enc(
        x, w1eff, _b(ne[0]), ne[1]["W"], _b(ne[1]), ne[2]["W"], _b(ne[2]),
        prm["node_enc_ln"]["g"].reshape(1, -1),
        prm["node_enc_ln"]["b"].reshape(1, -1), wpq0, n)

    zeros_n = jnp.zeros(((-(-n // _NS) + 7) // 8 * 8, _L), jnp.float32)
    gather128 = _pair_gather(e2, _L, +1, c=40)
    scatter = _scatter_add(n, e2, _L, c=40)

    for i in range(len(steps)):
        sp = steps[i]
        em = sp["edge_mlp"]
        nm = sp["node_mlp"]
        _, _, w1c = _split_edge_w1(i)
        eln_g = sp["edge_ln"]["g"].reshape(1, -1)
        eln_b = sp["edge_ln"]["b"].reshape(1, -1)
        gg_h = [gather128(p_tab, q_tab, s_h[0], r_h[0]),
                gather128(p_tab, q_tab, s_h[1], r_h[1])]
        hen_h = []
        for hlf in range(2):
            heo, hen = _call_edge_step(
                gg_h[hlf], he_h[hlf], w1c, _b(em[0]), em[1]["W"], _b(em[1]),
                em[2]["W"], _b(em[2]), eln_g, eln_b, e2, be=be2)
            he_h[hlf] = heo
            hen_h.append(hen)
        agg2a = scatter(hen_h[0], r_h[0], zeros_n)
        agg2b = scatter(hen_h[1], r_h[1], zeros_n)
        wn1 = nm[0]["W"]
        wh, wa = wn1[:_L], wn1[_L:]
        lng = sp["node_ln"]["g"].reshape(1, -1)
        lnb = sp["node_ln"]["b"].reshape(1, -1)
        if i + 1 < len(steps):
            w1s, w1r, _ = _split_edge_w1(i + 1)
            wpq = jnp.concatenate([w1s, w1r], axis=1)
            h, p_tab, q_tab = _call_node_step(
                h, agg2a, agg2b, wh, wa, _b(nm[0]), nm[1]["W"], _b(nm[1]),
                nm[2]["W"], _b(nm[2]), lng, lnb, wpq, n)
        else:
            h = _call_node_last(
                h, agg2a, agg2b, wh, wa, _b(nm[0]), nm[1]["W"], _b(nm[1]),
                nm[2]["W"], _b(nm[2]), lng, lnb, n)

    dec = prm["decoder"]
    w3p = jnp.pad(dec[2]["W"], ((0, 0), (0, _L - 3)))
    b3p = jnp.pad(dec[2]["b"], (0, _L - 3)).reshape(1, -1)
    acc = _call_dec(h, dec[0]["W"], _b(dec[0]), dec[1]["W"], _b(dec[1]),
                    w3p, b3p, n)[:, :3]

    prev_vel = position_sequence[:, -1] - position_sequence[:, -2]
    return position_sequence[:, -1] + prev_vel + acc


# speedup vs baseline: 4.6430x; 1.2677x over previous
import functools

import jax
import jax.numpy as jnp
import numpy as np
from jax import lax
from jax.experimental import pallas as pl
from jax.experimental.pallas import tpu as pltpu
from jax.experimental.pallas import tpu_sc as plsc

RADIUS = 0.015
_BOUNDS = np.array([[0.1, 0.9], [0.1, 0.9], [0.1, 0.9]], dtype=np.float32)
_EPS = 1e-5
_NC, _NS = 2, 16
_NW = _NC * _NS
_C = 40
_BN = 1000
_BE = 2560
_L = 128


def _pair_gather(n_edges, d, sign, c=80):
    ew = n_edges // _NW
    nch = ew // c
    assert ew * _NW == n_edges and nch * c == ew and nch >= 2
    assert c % 8 == 0 and c <= 128
    mesh = plsc.VectorSubcoreMesh(core_axis_name="c", subcore_axis_name="s")

    def body(a_hbm, b_hbm, s_hbm, r_hbm, out_hbm,
             si0, ri0, si1, ri1, bp0, bq0, bp1, bq1, sa0, sb0, sa1, sb1):
        wid = lax.axis_index("s") * _NC + lax.axis_index("c")
        base = pl.multiple_of(wid * ew, 8)

        def load_fire(j, si, ri, bp, bq, sa, sb):
            off = pl.multiple_of(base + j * c, 8)
            pltpu.sync_copy(s_hbm.at[pl.ds(off, c)], si)
            pltpu.sync_copy(r_hbm.at[pl.ds(off, c)], ri)
            pltpu.async_copy(a_hbm.at[si], bp, sa)
            pltpu.async_copy(b_hbm.at[ri], bq, sb)

        def wait(si, ri, bp, bq, sa, sb):
            pltpu.make_async_copy(a_hbm.at[si], bp, sa).wait()
            pltpu.make_async_copy(b_hbm.at[ri], bq, sb).wait()

        def combine_store(j, bp, bq):
            def row(i, cc):
                for g in range(d // 16):
                    sl = pl.ds(g * 16, 16)
                    if sign > 0:
                        bp[i, sl] = bp[i, sl] + bq[i, sl]
                    else:
                        bp[i, sl] = bp[i, sl] - bq[i, sl]
                return cc
            lax.fori_loop(0, c, row, 0)
            off = pl.multiple_of(base + j * c, 8)
            pltpu.sync_copy(bp, out_hbm.at[pl.ds(off, c)])

        load_fire(0, si0, ri0, bp0, bq0, sa0, sb0)
        load_fire(1, si1, ri1, bp1, bq1, sa1, sb1)

        def pair(p, cc):
            j0 = 2 * p
            wait(si0, ri0, bp0, bq0, sa0, sb0)
            combine_store(j0, bp0, bq0)

            @pl.when(j0 + 2 < nch)
            def _():
                load_fire(j0 + 2, si0, ri0, bp0, bq0, sa0, sb0)

            wait(si1, ri1, bp1, bq1, sa1, sb1)
            combine_store(j0 + 1, bp1, bq1)

            @pl.when(j0 + 3 < nch)
            def _():
                load_fire(j0 + 3, si1, ri1, bp1, bq1, sa1, sb1)
            return cc

        lax.fori_loop(0, nch // 2, pair, 0)
        if nch % 2 == 1:
            wait(si0, ri0, bp0, bq0, sa0, sb0)
            combine_store(nch - 1, bp0, bq0)

    return functools.partial(
        pl.kernel, body,
        out_type=jax.ShapeDtypeStruct((n_edges, d), jnp.float32),
        mesh=mesh,
        compiler_params=pltpu.CompilerParams(use_tc_tiling_on_sc=(d % 128 == 0)),
        scratch_types=[
            pltpu.VMEM((c,), jnp.int32), pltpu.VMEM((c,), jnp.int32),
            pltpu.VMEM((c,), jnp.int32), pltpu.VMEM((c,), jnp.int32),
            pltpu.VMEM((c, d), jnp.float32), pltpu.VMEM((c, d), jnp.float32),
            pltpu.VMEM((c, d), jnp.float32), pltpu.VMEM((c, d), jnp.float32),
            pltpu.SemaphoreType.DMA, pltpu.SemaphoreType.DMA,
            pltpu.SemaphoreType.DMA, pltpu.SemaphoreType.DMA,
        ],
    )()


def _scatter_add(n_nodes, n_edges, d, c=80):
    ew = n_edges // _NW
    nch = ew // c
    rt = (-(-n_nodes // _NS) + 7) // 8 * 8
    n_pad = rt * _NS
    last = n_nodes - rt * (_NS - 1)
    assert nch * c == ew and nch >= 2
    assert last > 0 and last % 8 == 0 and n_nodes % 8 == 0
    mesh = plsc.VectorSubcoreMesh(core_axis_name="c", subcore_axis_name="s")

    def body(v_hbm, r_hbm, z_hbm, out_hbm, ri0, ri1, b0, b1, sa0, sa1, shared):
        cid = lax.axis_index("c")
        sid = lax.axis_index("s")
        wid = sid * _NC + cid
        base = pl.multiple_of(wid * ew, 8)
        roff = pl.multiple_of(sid * rt, 8)

        pltpu.sync_copy(z_hbm, shared.at[pl.ds(roff, rt)])
        plsc.subcore_barrier()

        def load(j, ri, b, sa):
            off = pl.multiple_of(base + j * c, 8)
            pltpu.sync_copy(r_hbm.at[pl.ds(off, c)], ri)
            pltpu.async_copy(v_hbm.at[pl.ds(off, c)], b, sa)

        def wait(b, sa):
            pltpu.make_async_copy(v_hbm.at[pl.ds(0, c)], b, sa).wait()

        load(0, ri0, b0, sa0)
        load(1, ri1, b1, sa1)

        def pair(p, cc):
            j0 = 2 * p
            wait(b0, sa0)
            pltpu.sync_copy(b0, shared.at[ri0], add=True)

            @pl.when(j0 + 2 < nch)
            def _():
                load(j0 + 2, ri0, b0, sa0)

            wait(b1, sa1)
            pltpu.sync_copy(b1, shared.at[ri1], add=True)

            @pl.when(j0 + 3 < nch)
            def _():
                load(j0 + 3, ri1, b1, sa1)
            return cc

        lax.fori_loop(0, nch // 2, pair, 0)
        if nch % 2 == 1:
            wait(b0, sa0)
            pltpu.sync_copy(b0, shared.at[ri0], add=True)

        plsc.subcore_barrier()
        obase = pl.multiple_of(cid * n_nodes + roff, 8)

        @pl.when(sid == _NS - 1)
        def _dump_last():
            pltpu.sync_copy(shared.at[pl.ds(roff, last)],
                            out_hbm.at[pl.ds(obase, last)])

        @pl.when(sid < _NS - 1)
        def _dump_full():
            pltpu.sync_copy(shared.at[pl.ds(roff, rt)],
                            out_hbm.at[pl.ds(obase, rt)])

    return functools.partial(
        pl.kernel, body,
        out_type=jax.ShapeDtypeStruct((2 * n_nodes, d), jnp.float32),
        mesh=mesh,
        scratch_types=[
            pltpu.VMEM((c,), jnp.int32), pltpu.VMEM((c,), jnp.int32),
            pltpu.VMEM((c, d), jnp.float32), pltpu.VMEM((c, d), jnp.float32),
            pltpu.SemaphoreType.DMA, pltpu.SemaphoreType.DMA,
            pltpu.VMEM_SHARED((n_pad, d), jnp.float32),
        ],
    )()


def _ln_in(v, g, b):
    m = jnp.mean(v, axis=-1, keepdims=True)
    var = jnp.mean((v - m) ** 2, axis=-1, keepdims=True)
    return (v - m) / jnp.sqrt(var + _EPS) * g + b


def _dot(a, b):
    return jnp.dot(a, b, preferred_element_type=jnp.float32)


def _wspec():
    return pl.BlockSpec((_L, _L), lambda i: (0, 0))


def _bspec():
    return pl.BlockSpec((1, _L), lambda i: (0, 0))


def _node_enc_body(x_ref, w1, b1, w2, b2, w3, b3, g, bb, wpq,
                   h_ref, p_ref, q_ref):
    t = jnp.maximum(_dot(x_ref[...], w1[...]) + b1[...], 0.0)
    t = jnp.maximum(_dot(t, w2[...]) + b2[...], 0.0)
    v = _dot(t, w3[...]) + b3[...]
    h = _ln_in(v, g[...], bb[...])
    h_ref[...] = h
    pq = _dot(h, wpq[...])
    p_ref[...] = pq[:, :_L]
    q_ref[...] = pq[:, _L:]


def _call_node_enc(x, w1, b1, w2, b2, w3, b3, g, bb, wpq, n):
    grid = (n // _BN,)
    row = pl.BlockSpec((_BN, _L), lambda i: (i, 0))
    return pl.pallas_call(
        _node_enc_body,
        grid=grid,
        in_specs=[row, _wspec(), _bspec(), _wspec(), _bspec(), _wspec(),
                  _bspec(), _bspec(), _bspec(),
                  pl.BlockSpec((_L, 2 * _L), lambda i: (0, 0))],
        out_specs=[row, row, row],
        out_shape=[jax.ShapeDtypeStruct((n, _L), jnp.float32)] * 3,
    )(x, w1, b1, w2, b2, w3, b3, g, bb, wpq)


def _edge_enc_body(rr_ref, w1p, w1d, b1, w2, b2, w3, b3, g, bb, he_ref):
    u = rr_ref[...] * (1.0 / RADIUS)
    dist = jnp.sqrt(jnp.sum(u * u, axis=-1, keepdims=True))
    t = jnp.maximum(_dot(u, w1p[...]) + dist * w1d[...] + b1[...], 0.0)
    t = jnp.maximum(_dot(t, w2[...]) + b2[...], 0.0)
    v = _dot(t, w3[...]) + b3[...]
    he_ref[...] = _ln_in(v, g[...], bb[...])


def _call_edge_enc(rr, w1p, w1d, b1, w2, b2, w3, b3, g, bb, e, be=_BE):
    grid = (e // be,)
    row16 = pl.BlockSpec((be, 16), lambda i: (i, 0))
    row = pl.BlockSpec((be, _L), lambda i: (i, 0))
    return pl.pallas_call(
        _edge_enc_body,
        grid=grid,
        in_specs=[row16, pl.BlockSpec((16, _L), lambda i: (0, 0)), _bspec(),
                  _bspec(), _wspec(), _bspec(), _wspec(), _bspec(),
                  _bspec(), _bspec()],
        out_specs=row,
        out_shape=jax.ShapeDtypeStruct((e, _L), jnp.float32),
    )(rr, w1p, w1d, b1, w2, b2, w3, b3, g, bb)


def _edge_step_body(g_ref, he_ref, w1, b1, w2, b2, w3, b3, g, bb,
                    heo_ref, hen_ref):
    he = he_ref[...]
    t = jnp.maximum(g_ref[...] + _dot(he, w1[...]) + b1[...], 0.0)
    t = jnp.maximum(_dot(t, w2[...]) + b2[...], 0.0)
    v = _dot(t, w3[...]) + b3[...]
    hn = _ln_in(v, g[...], bb[...])
    hen_ref[...] = hn
    heo_ref[...] = he + hn


def _call_edge_step(gg, he, w1, b1, w2, b2, w3, b3, g, bb, e, be=_BE):
    grid = (e // be,)
    row = pl.BlockSpec((be, _L), lambda i: (i, 0))
    return pl.pallas_call(
        _edge_step_body,
        grid=grid,
        in_specs=[row, row, _wspec(), _bspec(), _wspec(), _bspec(),
                  _wspec(), _bspec(), _bspec(), _bspec()],
        out_specs=[row, row],
        out_shape=[jax.ShapeDtypeStruct((e, _L), jnp.float32)] * 2,
    )(gg, he, w1, b1, w2, b2, w3, b3, g, bb)


def _node_step_body(h_ref, aggA_ref, aggB_ref, aggC_ref, aggD_ref,
                    wh, wa, b1, w2, b2, w3, b3,
                    g, bb, wpq, ho_ref, p_ref, q_ref):
    h = h_ref[...]
    agg = (aggA_ref[...] + aggB_ref[...]) + (aggC_ref[...] + aggD_ref[...])
    t = jnp.maximum(_dot(h, wh[...]) + _dot(agg, wa[...]) + b1[...], 0.0)
    t = jnp.maximum(_dot(t, w2[...]) + b2[...], 0.0)
    v = _dot(t, w3[...]) + b3[...]
    ho = h + _ln_in(v, g[...], bb[...])
    ho_ref[...] = ho
    pq = _dot(ho, wpq[...])
    p_ref[...] = pq[:, :_L]
    q_ref[...] = pq[:, _L:]


def _call_node_step(h, agg2a, agg2b, wh, wa, b1, w2, b2, w3, b3, g, bb, wpq, n):
    grid = (n // _BN,)
    row = pl.BlockSpec((_BN, _L), lambda i: (i, 0))
    rowB = pl.BlockSpec((_BN, _L), lambda i: (i + n // _BN, 0))
    return pl.pallas_call(
        _node_step_body,
        grid=grid,
        in_specs=[row, row, rowB, row, rowB, _wspec(), _wspec(), _bspec(),
                  _wspec(), _bspec(), _wspec(), _bspec(), _bspec(), _bspec(),
                  pl.BlockSpec((_L, 2 * _L), lambda i: (0, 0))],
        out_specs=[row, row, row],
        out_shape=[jax.ShapeDtypeStruct((n, _L), jnp.float32)] * 3,
    )(h, agg2a, agg2a, agg2b, agg2b, wh, wa, b1, w2, b2, w3, b3, g, bb, wpq)


def _node_last_body(h_ref, aggA_ref, aggB_ref, aggC_ref, aggD_ref,
                    wh, wa, b1, w2, b2, w3, b3, g, bb, ho_ref):
    h = h_ref[...]
    agg = (aggA_ref[...] + aggB_ref[...]) + (aggC_ref[...] + aggD_ref[...])
    t = jnp.maximum(_dot(h, wh[...]) + _dot(agg, wa[...]) + b1[...], 0.0)
    t = jnp.maximum(_dot(t, w2[...]) + b2[...], 0.0)
    v = _dot(t, w3[...]) + b3[...]
    ho_ref[...] = h + _ln_in(v, g[...], bb[...])


def _call_node_last(h, agg2a, agg2b, wh, wa, b1, w2, b2, w3, b3, g, bb, n):
    grid = (n // _BN,)
    row = pl.BlockSpec((_BN, _L), lambda i: (i, 0))
    rowB = pl.BlockSpec((_BN, _L), lambda i: (i + n // _BN, 0))
    return pl.pallas_call(
        _node_last_body,
        grid=grid,
        in_specs=[row, row, rowB, row, rowB, _wspec(), _wspec(), _bspec(),
                  _wspec(), _bspec(), _wspec(), _bspec(), _bspec(), _bspec()],
        out_specs=row,
        out_shape=jax.ShapeDtypeStruct((n, _L), jnp.float32),
    )(h, agg2a, agg2a, agg2b, agg2b, wh, wa, b1, w2, b2, w3, b3, g, bb)


def _dec_body(h_ref, w1, b1, w2, b2, w3p, b3p, acc_ref):
    t = jnp.maximum(_dot(h_ref[...], w1[...]) + b1[...], 0.0)
    t = jnp.maximum(_dot(t, w2[...]) + b2[...], 0.0)
    acc_ref[...] = _dot(t, w3p[...]) + b3p[...]


def _call_dec(h, w1, b1, w2, b2, w3p, b3p, n):
    grid = (n // _BN,)
    row = pl.BlockSpec((_BN, _L), lambda i: (i, 0))
    return pl.pallas_call(
        _dec_body,
        grid=grid,
        in_specs=[row, _wspec(), _bspec(), _wspec(), _bspec(), _wspec(),
                  _bspec()],
        out_specs=row,
        out_shape=jax.ShapeDtypeStruct((n, _L), jnp.float32),
    )(h, w1, b1, w2, b2, w3p, b3p)


def _b(p):
    return p["b"].reshape(1, -1)


def kernel(position_sequence, particle_types, edge_index, n_particles_per_example, params):
    n = position_sequence.shape[0]
    e = edge_index.shape[1]
    boundaries = jnp.asarray(_BOUNDS)
    most_recent = position_sequence[:, -1]
    vel = position_sequence[:, 1:] - position_sequence[:, :-1]
    flat_vel = vel.reshape(n, -1)
    d_lo = most_recent - boundaries[:, 0][None]
    d_hi = boundaries[:, 1][None] - most_recent
    d_b = jnp.clip(jnp.concatenate([d_lo, d_hi], axis=1) / RADIUS, -1.0, 1.0)
    onehot = jax.nn.one_hot(particle_types, 9, dtype=jnp.float32)
    x = jnp.pad(jnp.concatenate([flat_vel, d_b, onehot], axis=1),
                ((0, 0), (0, _L - 30)))
    senders = edge_index[0]
    receivers = edge_index[1]

    prm = params
    steps = prm["steps"]

    ne = prm["node_enc"]
    w1n = ne[0]["W"]
    w1eff = jnp.concatenate([w1n[:21], prm["type_emb"] @ w1n[21:37]], axis=0)
    w1eff = jnp.pad(w1eff, ((0, _L - 30), (0, 0)))

    def _split_edge_w1(i):
        w = steps[i]["edge_mlp"][0]["W"]
        return w[:_L], w[_L:2 * _L], w[2 * _L:]

    ee = prm["edge_enc"]
    w1e = ee[0]["W"]
    w1p = jnp.pad(w1e[:3], ((0, 13), (0, 0)))
    w1d = w1e[3].reshape(1, -1)

    assert e % 2560 == 0 and e >= 2 * 5120
    eA = -(-(e // 2) // 5120) * 5120
    eh = [eA, e - eA]
    be2 = 2560
    s_h = [senders[:eA], senders[eA:]]
    r_h = [receivers[:eA], receivers[eA:]]

    tpos = jnp.pad(most_recent, ((0, 0), (0, 13)))
    enc_ln_g = prm["edge_enc_ln"]["g"].reshape(1, -1)
    enc_ln_b = prm["edge_enc_ln"]["b"].reshape(1, -1)
    he_h = []
    for hlf in range(2):
        rr = _pair_gather(eh[hlf], 16, -1, c=80)(
            tpos, tpos, s_h[hlf], r_h[hlf])
        he_h.append(_call_edge_enc(
            rr, w1p, w1d, _b(ee[0]), ee[1]["W"], _b(ee[1]), ee[2]["W"],
            _b(ee[2]), enc_ln_g, enc_ln_b, eh[hlf], be=be2))

    w1s0, w1r0, _ = _split_edge_w1(0)
    wpq0 = jnp.concatenate([w1s0, w1r0], axis=1)
    h, p_tab, q_tab = _call_node_enc(
        x, w1eff, _b(ne[0]), ne[1]["W"], _b(ne[1]), ne[2]["W"], _b(ne[2]),
        prm["node_enc_ln"]["g"].reshape(1, -1),
        prm["node_enc_ln"]["b"].reshape(1, -1), wpq0, n)

    zeros_n = jnp.zeros(((-(-n // _NS) + 7) // 8 * 8, _L), jnp.float32)
    gath_h = [_pair_gather(eh[0], _L, +1, c=80),
              _pair_gather(eh[1], _L, +1, c=80)]
    scat_h = [_scatter_add(n, eh[0], _L, c=80),
              _scatter_add(n, eh[1], _L, c=80)]

    for i in range(len(steps)):
        sp = steps[i]
        em = sp["edge_mlp"]
        nm = sp["node_mlp"]
        _, _, w1c = _split_edge_w1(i)
        eln_g = sp["edge_ln"]["g"].reshape(1, -1)
        eln_b = sp["edge_ln"]["b"].reshape(1, -1)
        gg_h = [gath_h[0](p_tab, q_tab, s_h[0], r_h[0]),
                gath_h[1](p_tab, q_tab, s_h[1], r_h[1])]
        hen_h = []
        for hlf in range(2):
            heo, hen = _call_edge_step(
                gg_h[hlf], he_h[hlf], w1c, _b(em[0]), em[1]["W"], _b(em[1]),
                em[2]["W"], _b(em[2]), eln_g, eln_b, eh[hlf], be=be2)
            he_h[hlf] = heo
            hen_h.append(hen)
        agg2a = scat_h[0](hen_h[0], r_h[0], zeros_n)
        agg2b = scat_h[1](hen_h[1], r_h[1], zeros_n)
        wn1 = nm[0]["W"]
        wh, wa = wn1[:_L], wn1[_L:]
        lng = sp["node_ln"]["g"].reshape(1, -1)
        lnb = sp["node_ln"]["b"].reshape(1, -1)
        if i + 1 < len(steps):
            w1s, w1r, _ = _split_edge_w1(i + 1)
            wpq = jnp.concatenate([w1s, w1r], axis=1)
            h, p_tab, q_tab = _call_node_step(
                h, agg2a, agg2b, wh, wa, _b(nm[0]), nm[1]["W"], _b(nm[1]),
                nm[2]["W"], _b(nm[2]), lng, lnb, wpq, n)
        else:
            h = _call_node_last(
                h, agg2a, agg2b, wh, wa, _b(nm[0]), nm[1]["W"], _b(nm[1]),
                nm[2]["W"], _b(nm[2]), lng, lnb, n)

    dec = prm["decoder"]
    w3p = jnp.pad(dec[2]["W"], ((0, 0), (0, _L - 3)))
    b3p = jnp.pad(dec[2]["b"], (0, _L - 3)).reshape(1, -1)
    acc = _call_dec(h, dec[0]["W"], _b(dec[0]), dec[1]["W"], _b(dec[1]),
                    w3p, b3p, n)[:, :3]

    prev_vel = position_sequence[:, -1] - position_sequence[:, -2]
    return position_sequence[:, -1] + prev_vel + acc


# speedup vs baseline: 4.7479x; 1.0226x over previous
import functools

import jax
import jax.numpy as jnp
import numpy as np
from jax import lax
from jax.experimental import pallas as pl
from jax.experimental.pallas import tpu as pltpu
from jax.experimental.pallas import tpu_sc as plsc

RADIUS = 0.015
_BOUNDS = np.array([[0.1, 0.9], [0.1, 0.9], [0.1, 0.9]], dtype=np.float32)
_EPS = 1e-5
_NC, _NS = 2, 16
_NW = _NC * _NS
_C = 40
_BN = 1000
_BE = 2560
_L = 128


def _pair_gather(n_edges, d, sign, c=80):
    ew = n_edges // _NW
    nch = ew // c
    assert ew * _NW == n_edges and nch * c == ew and nch >= 2
    assert c % 8 == 0 and c <= 128
    mesh = plsc.VectorSubcoreMesh(core_axis_name="c", subcore_axis_name="s")

    def body(a_hbm, b_hbm, s_hbm, r_hbm, out_hbm,
             si0, ri0, si1, ri1, bp0, bq0, bp1, bq1, sa0, sb0, sa1, sb1):
        wid = lax.axis_index("s") * _NC + lax.axis_index("c")
        base = pl.multiple_of(wid * ew, 8)

        def load_fire(j, si, ri, bp, bq, sa, sb):
            off = pl.multiple_of(base + j * c, 8)
            pltpu.sync_copy(s_hbm.at[pl.ds(off, c)], si)
            pltpu.sync_copy(r_hbm.at[pl.ds(off, c)], ri)
            pltpu.async_copy(a_hbm.at[si], bp, sa)
            pltpu.async_copy(b_hbm.at[ri], bq, sb)

        def wait(si, ri, bp, bq, sa, sb):
            pltpu.make_async_copy(a_hbm.at[si], bp, sa).wait()
            pltpu.make_async_copy(b_hbm.at[ri], bq, sb).wait()

        def combine_store(j, bp, bq):
            def row(i, cc):
                for g in range(d // 16):
                    sl = pl.ds(g * 16, 16)
                    if sign > 0:
                        bp[i, sl] = bp[i, sl] + bq[i, sl]
                    else:
                        bp[i, sl] = bp[i, sl] - bq[i, sl]
                return cc
            lax.fori_loop(0, c, row, 0)
            off = pl.multiple_of(base + j * c, 8)
            pltpu.sync_copy(bp, out_hbm.at[pl.ds(off, c)])

        load_fire(0, si0, ri0, bp0, bq0, sa0, sb0)
        load_fire(1, si1, ri1, bp1, bq1, sa1, sb1)

        def pair(p, cc):
            j0 = 2 * p
            wait(si0, ri0, bp0, bq0, sa0, sb0)
            combine_store(j0, bp0, bq0)

            @pl.when(j0 + 2 < nch)
            def _():
                load_fire(j0 + 2, si0, ri0, bp0, bq0, sa0, sb0)

            wait(si1, ri1, bp1, bq1, sa1, sb1)
            combine_store(j0 + 1, bp1, bq1)

            @pl.when(j0 + 3 < nch)
            def _():
                load_fire(j0 + 3, si1, ri1, bp1, bq1, sa1, sb1)
            return cc

        lax.fori_loop(0, nch // 2, pair, 0)
        if nch % 2 == 1:
            wait(si0, ri0, bp0, bq0, sa0, sb0)
            combine_store(nch - 1, bp0, bq0)

    return functools.partial(
        pl.kernel, body,
        out_type=jax.ShapeDtypeStruct((n_edges, d), jnp.float32),
        mesh=mesh,
        compiler_params=pltpu.CompilerParams(use_tc_tiling_on_sc=(d % 128 == 0)),
        scratch_types=[
            pltpu.VMEM((c,), jnp.int32), pltpu.VMEM((c,), jnp.int32),
            pltpu.VMEM((c,), jnp.int32), pltpu.VMEM((c,), jnp.int32),
            pltpu.VMEM((c, d), jnp.float32), pltpu.VMEM((c, d), jnp.float32),
            pltpu.VMEM((c, d), jnp.float32), pltpu.VMEM((c, d), jnp.float32),
            pltpu.SemaphoreType.DMA, pltpu.SemaphoreType.DMA,
            pltpu.SemaphoreType.DMA, pltpu.SemaphoreType.DMA,
        ],
    )()


def _scatter_add(n_nodes, n_edges, d, c=80):
    ew = n_edges // _NW
    nch = ew // c
    rt = (-(-n_nodes // _NS) + 7) // 8 * 8
    n_pad = rt * _NS
    last = n_nodes - rt * (_NS - 1)
    assert nch * c == ew and nch >= 2
    assert last > 0 and last % 8 == 0 and n_nodes % 8 == 0
    mesh = plsc.VectorSubcoreMesh(core_axis_name="c", subcore_axis_name="s")

    def body(v_hbm, r_hbm, z_hbm, out_hbm, ri0, ri1, b0, b1, sa0, sa1, shared):
        cid = lax.axis_index("c")
        sid = lax.axis_index("s")
        wid = sid * _NC + cid
        base = pl.multiple_of(wid * ew, 8)
        roff = pl.multiple_of(sid * rt, 8)

        pltpu.sync_copy(z_hbm, shared.at[pl.ds(roff, rt)])
        plsc.subcore_barrier()

        def load(j, ri, b, sa):
            off = pl.multiple_of(base + j * c, 8)
            pltpu.sync_copy(r_hbm.at[pl.ds(off, c)], ri)
            pltpu.async_copy(v_hbm.at[pl.ds(off, c)], b, sa)

        def wait(b, sa):
            pltpu.make_async_copy(v_hbm.at[pl.ds(0, c)], b, sa).wait()

        load(0, ri0, b0, sa0)
        load(1, ri1, b1, sa1)

        def pair(p, cc):
            j0 = 2 * p
            wait(b0, sa0)
            pltpu.sync_copy(b0, shared.at[ri0], add=True)

            @pl.when(j0 + 2 < nch)
            def _():
                load(j0 + 2, ri0, b0, sa0)

            wait(b1, sa1)
            pltpu.sync_copy(b1, shared.at[ri1], add=True)

            @pl.when(j0 + 3 < nch)
            def _():
                load(j0 + 3, ri1, b1, sa1)
            return cc

        lax.fori_loop(0, nch // 2, pair, 0)
        if nch % 2 == 1:
            wait(b0, sa0)
            pltpu.sync_copy(b0, shared.at[ri0], add=True)

        plsc.subcore_barrier()
        obase = pl.multiple_of(cid * n_nodes + roff, 8)

        @pl.when(sid == _NS - 1)
        def _dump_last():
            pltpu.sync_copy(shared.at[pl.ds(roff, last)],
                            out_hbm.at[pl.ds(obase, last)])

        @pl.when(sid < _NS - 1)
        def _dump_full():
            pltpu.sync_copy(shared.at[pl.ds(roff, rt)],
                            out_hbm.at[pl.ds(obase, rt)])

    return functools.partial(
        pl.kernel, body,
        out_type=jax.ShapeDtypeStruct((2 * n_nodes, d), jnp.float32),
        mesh=mesh,
        scratch_types=[
            pltpu.VMEM((c,), jnp.int32), pltpu.VMEM((c,), jnp.int32),
            pltpu.VMEM((c, d), jnp.float32), pltpu.VMEM((c, d), jnp.float32),
            pltpu.SemaphoreType.DMA, pltpu.SemaphoreType.DMA,
            pltpu.VMEM_SHARED((n_pad, d), jnp.float32),
        ],
    )()


def _ln_in(v, g, b):
    m = jnp.mean(v, axis=-1, keepdims=True)
    var = jnp.mean((v - m) ** 2, axis=-1, keepdims=True)
    return (v - m) / jnp.sqrt(var + _EPS) * g + b


def _dot(a, b):
    return jnp.dot(a, b, preferred_element_type=jnp.float32)


def _wspec():
    return pl.BlockSpec((_L, _L), lambda i: (0, 0))


def _bspec():
    return pl.BlockSpec((1, _L), lambda i: (0, 0))


def _node_enc_body(x_ref, w1, b1, w2, b2, w3, b3, g, bb, wpq,
                   h_ref, p_ref, q_ref):
    t = jnp.maximum(_dot(x_ref[...], w1[...]) + b1[...], 0.0)
    t = jnp.maximum(_dot(t, w2[...]) + b2[...], 0.0)
    v = _dot(t, w3[...]) + b3[...]
    h = _ln_in(v, g[...], bb[...])
    h_ref[...] = h
    pq = _dot(h, wpq[...])
    p_ref[...] = pq[:, :_L]
    q_ref[...] = pq[:, _L:]


def _call_node_enc(x, w1, b1, w2, b2, w3, b3, g, bb, wpq, n):
    grid = (n // _BN,)
    row = pl.BlockSpec((_BN, _L), lambda i: (i, 0))
    return pl.pallas_call(
        _node_enc_body,
        grid=grid,
        in_specs=[row, _wspec(), _bspec(), _wspec(), _bspec(), _wspec(),
                  _bspec(), _bspec(), _bspec(),
                  pl.BlockSpec((_L, 2 * _L), lambda i: (0, 0))],
        out_specs=[row, row, row],
        out_shape=[jax.ShapeDtypeStruct((n, _L), jnp.float32)] * 3,
    )(x, w1, b1, w2, b2, w3, b3, g, bb, wpq)


def _edge_enc_body(rr_ref, w1p, w1d, b1, w2, b2, w3, b3, g, bb, he_ref):
    u = rr_ref[...] * (1.0 / RADIUS)
    dist = jnp.sqrt(jnp.sum(u * u, axis=-1, keepdims=True))
    t = jnp.maximum(_dot(u, w1p[...]) + dist * w1d[...] + b1[...], 0.0)
    t = jnp.maximum(_dot(t, w2[...]) + b2[...], 0.0)
    v = _dot(t, w3[...]) + b3[...]
    he_ref[...] = _ln_in(v, g[...], bb[...])


def _call_edge_enc(rr, w1p, w1d, b1, w2, b2, w3, b3, g, bb, e, be=_BE):
    grid = (e // be,)
    row16 = pl.BlockSpec((be, 16), lambda i: (i, 0))
    row = pl.BlockSpec((be, _L), lambda i: (i, 0))
    return pl.pallas_call(
        _edge_enc_body,
        grid=grid,
        in_specs=[row16, pl.BlockSpec((16, _L), lambda i: (0, 0)), _bspec(),
                  _bspec(), _wspec(), _bspec(), _wspec(), _bspec(),
                  _bspec(), _bspec()],
        out_specs=row,
        out_shape=jax.ShapeDtypeStruct((e, _L), jnp.float32),
    )(rr, w1p, w1d, b1, w2, b2, w3, b3, g, bb)


def _edge_step_body(g_ref, he_ref, w1, b1, w2, b2, w3, b3, g, bb,
                    heo_ref, hen_ref):
    he = he_ref[...]
    t = jnp.maximum(g_ref[...] + _dot(he, w1[...]) + b1[...], 0.0)
    t = jnp.maximum(_dot(t, w2[...]) + b2[...], 0.0)
    v = _dot(t, w3[...]) + b3[...]
    hn = _ln_in(v, g[...], bb[...])
    hen_ref[...] = hn
    heo_ref[...] = he + hn


def _call_edge_step(gg, he, w1, b1, w2, b2, w3, b3, g, bb, e, be=_BE):
    grid = (e // be,)
    row = pl.BlockSpec((be, _L), lambda i: (i, 0))
    return pl.pallas_call(
        _edge_step_body,
        grid=grid,
        in_specs=[row, row, _wspec(), _bspec(), _wspec(), _bspec(),
                  _wspec(), _bspec(), _bspec(), _bspec()],
        out_specs=[row, row],
        out_shape=[jax.ShapeDtypeStruct((e, _L), jnp.float32)] * 2,
    )(gg, he, w1, b1, w2, b2, w3, b3, g, bb)


def _node_step_body(h_ref, aggA_ref, aggB_ref, aggC_ref, aggD_ref,
                    wh, wa, b1, w2, b2, w3, b3,
                    g, bb, wpq, ho_ref, p_ref, q_ref):
    h = h_ref[...]
    agg = (aggA_ref[...] + aggB_ref[...]) + (aggC_ref[...] + aggD_ref[...])
    t = jnp.maximum(_dot(h, wh[...]) + _dot(agg, wa[...]) + b1[...], 0.0)
    t = jnp.maximum(_dot(t, w2[...]) + b2[...], 0.0)
    v = _dot(t, w3[...]) + b3[...]
    ho = h + _ln_in(v, g[...], bb[...])
    ho_ref[...] = ho
    pq = _dot(ho, wpq[...])
    p_ref[...] = pq[:, :_L]
    q_ref[...] = pq[:, _L:]


def _call_node_step(h, agg2a, agg2b, wh, wa, b1, w2, b2, w3, b3, g, bb, wpq, n):
    grid = (n // _BN,)
    row = pl.BlockSpec((_BN, _L), lambda i: (i, 0))
    rowB = pl.BlockSpec((_BN, _L), lambda i: (i + n // _BN, 0))
    return pl.pallas_call(
        _node_step_body,
        grid=grid,
        in_specs=[row, row, rowB, row, rowB, _wspec(), _wspec(), _bspec(),
                  _wspec(), _bspec(), _wspec(), _bspec(), _bspec(), _bspec(),
                  pl.BlockSpec((_L, 2 * _L), lambda i: (0, 0))],
        out_specs=[row, row, row],
        out_shape=[jax.ShapeDtypeStruct((n, _L), jnp.float32)] * 3,
    )(h, agg2a, agg2a, agg2b, agg2b, wh, wa, b1, w2, b2, w3, b3, g, bb, wpq)


def _node_last_body(h_ref, aggA_ref, aggB_ref, aggC_ref, aggD_ref,
                    wh, wa, b1, w2, b2, w3, b3, g, bb, ho_ref):
    h = h_ref[...]
    agg = (aggA_ref[...] + aggB_ref[...]) + (aggC_ref[...] + aggD_ref[...])
    t = jnp.maximum(_dot(h, wh[...]) + _dot(agg, wa[...]) + b1[...], 0.0)
    t = jnp.maximum(_dot(t, w2[...]) + b2[...], 0.0)
    v = _dot(t, w3[...]) + b3[...]
    ho_ref[...] = h + _ln_in(v, g[...], bb[...])


def _call_node_last(h, agg2a, agg2b, wh, wa, b1, w2, b2, w3, b3, g, bb, n):
    grid = (n // _BN,)
    row = pl.BlockSpec((_BN, _L), lambda i: (i, 0))
    rowB = pl.BlockSpec((_BN, _L), lambda i: (i + n // _BN, 0))
    return pl.pallas_call(
        _node_last_body,
        grid=grid,
        in_specs=[row, row, rowB, row, rowB, _wspec(), _wspec(), _bspec(),
                  _wspec(), _bspec(), _wspec(), _bspec(), _bspec(), _bspec()],
        out_specs=row,
        out_shape=jax.ShapeDtypeStruct((n, _L), jnp.float32),
    )(h, agg2a, agg2a, agg2b, agg2b, wh, wa, b1, w2, b2, w3, b3, g, bb)


def _dec_body(h_ref, w1, b1, w2, b2, w3p, b3p, acc_ref):
    t = jnp.maximum(_dot(h_ref[...], w1[...]) + b1[...], 0.0)
    t = jnp.maximum(_dot(t, w2[...]) + b2[...], 0.0)
    acc_ref[...] = _dot(t, w3p[...]) + b3p[...]


def _call_dec(h, w1, b1, w2, b2, w3p, b3p, n):
    grid = (n // _BN,)
    row = pl.BlockSpec((_BN, _L), lambda i: (i, 0))
    return pl.pallas_call(
        _dec_body,
        grid=grid,
        in_specs=[row, _wspec(), _bspec(), _wspec(), _bspec(), _wspec(),
                  _bspec()],
        out_specs=row,
        out_shape=jax.ShapeDtypeStruct((n, _L), jnp.float32),
    )(h, w1, b1, w2, b2, w3p, b3p)


def _b(p):
    return p["b"].reshape(1, -1)


def kernel(position_sequence, particle_types, edge_index, n_particles_per_example, params):
    n = position_sequence.shape[0]
    e = edge_index.shape[1]
    boundaries = jnp.asarray(_BOUNDS)
    most_recent = position_sequence[:, -1]
    vel = position_sequence[:, 1:] - position_sequence[:, :-1]
    flat_vel = vel.reshape(n, -1)
    d_lo = most_recent - boundaries[:, 0][None]
    d_hi = boundaries[:, 1][None] - most_recent
    d_b = jnp.clip(jnp.concatenate([d_lo, d_hi], axis=1) / RADIUS, -1.0, 1.0)
    onehot = jax.nn.one_hot(particle_types, 9, dtype=jnp.float32)
    x = jnp.pad(jnp.concatenate([flat_vel, d_b, onehot], axis=1),
                ((0, 0), (0, _L - 30)))
    senders = edge_index[0]
    receivers = edge_index[1]

    prm = params
    steps = prm["steps"]

    ne = prm["node_enc"]
    w1n = ne[0]["W"]
    w1eff = jnp.concatenate([w1n[:21], prm["type_emb"] @ w1n[21:37]], axis=0)
    w1eff = jnp.pad(w1eff, ((0, _L - 30), (0, 0)))

    def _split_edge_w1(i):
        w = steps[i]["edge_mlp"][0]["W"]
        return w[:_L], w[_L:2 * _L], w[2 * _L:]

    ee = prm["edge_enc"]
    w1e = ee[0]["W"]
    w1p = jnp.pad(w1e[:3], ((0, 13), (0, 0)))
    w1d = w1e[3].reshape(1, -1)

    assert e % 2560 == 0 and e >= 2 * 5120
    eA = -(-(e // 2) // 5120) * 5120
    eh = [eA, e - eA]
    be2 = 2560
    s_h = [senders[:eA], senders[eA:]]
    r_h = [receivers[:eA], receivers[eA:]]

    tpos = jnp.pad(most_recent, ((0, 0), (0, 13)))
    enc_ln_g = prm["edge_enc_ln"]["g"].reshape(1, -1)
    enc_ln_b = prm["edge_enc_ln"]["b"].reshape(1, -1)
    he_h = []
    for hlf in range(2):
        rr = _pair_gather(eh[hlf], 16, -1, c=(128 if hlf == 0 else 80))(
            tpos, tpos, s_h[hlf], r_h[hlf])
        he_h.append(_call_edge_enc(
            rr, w1p, w1d, _b(ee[0]), ee[1]["W"], _b(ee[1]), ee[2]["W"],
            _b(ee[2]), enc_ln_g, enc_ln_b, eh[hlf], be=be2))

    w1s0, w1r0, _ = _split_edge_w1(0)
    wpq0 = jnp.concatenate([w1s0, w1r0], axis=1)
    h, p_tab, q_tab = _call_node_enc(
        x, w1eff, _b(ne[0]), ne[1]["W"], _b(ne[1]), ne[2]["W"], _b(ne[2]),
        prm["node_enc_ln"]["g"].reshape(1, -1),
        prm["node_enc_ln"]["b"].reshape(1, -1), wpq0, n)

    zeros_n = jnp.zeros(((-(-n // _NS) + 7) // 8 * 8, _L), jnp.float32)
    gath_h = [_pair_gather(eh[0], _L, +1, c=128),
              _pair_gather(eh[1], _L, +1, c=80)]
    scat_h = [_scatter_add(n, eh[0], _L, c=128),
              _scatter_add(n, eh[1], _L, c=80)]

    for i in range(len(steps)):
        sp = steps[i]
        em = sp["edge_mlp"]
        nm = sp["node_mlp"]
        _, _, w1c = _split_edge_w1(i)
        eln_g = sp["edge_ln"]["g"].reshape(1, -1)
        eln_b = sp["edge_ln"]["b"].reshape(1, -1)
        gg_h = [gath_h[0](p_tab, q_tab, s_h[0], r_h[0]),
                gath_h[1](p_tab, q_tab, s_h[1], r_h[1])]
        hen_h = []
        for hlf in range(2):
            heo, hen = _call_edge_step(
                gg_h[hlf], he_h[hlf], w1c, _b(em[0]), em[1]["W"], _b(em[1]),
                em[2]["W"], _b(em[2]), eln_g, eln_b, eh[hlf], be=be2)
            he_h[hlf] = heo
            hen_h.append(hen)
        agg2a = scat_h[0](hen_h[0], r_h[0], zeros_n)
        agg2b = scat_h[1](hen_h[1], r_h[1], zeros_n)
        wn1 = nm[0]["W"]
        wh, wa = wn1[:_L], wn1[_L:]
        lng = sp["node_ln"]["g"].reshape(1, -1)
        lnb = sp["node_ln"]["b"].reshape(1, -1)
        if i + 1 < len(steps):
            w1s, w1r, _ = _split_edge_w1(i + 1)
            wpq = jnp.concatenate([w1s, w1r], axis=1)
            h, p_tab, q_tab = _call_node_step(
                h, agg2a, agg2b, wh, wa, _b(nm[0]), nm[1]["W"], _b(nm[1]),
                nm[2]["W"], _b(nm[2]), lng, lnb, wpq, n)
        else:
            h = _call_node_last(
                h, agg2a, agg2b, wh, wa, _b(nm[0]), nm[1]["W"], _b(nm[1]),
                nm[2]["W"], _b(nm[2]), lng, lnb, n)

    dec = prm["decoder"]
    w3p = jnp.pad(dec[2]["W"], ((0, 0), (0, _L - 3)))
    b3p = jnp.pad(dec[2]["b"], (0, _L - 3)).reshape(1, -1)
    acc = _call_dec(h, dec[0]["W"], _b(dec[0]), dec[1]["W"], _b(dec[1]),
                    w3p, b3p, n)[:, :3]

    prev_vel = position_sequence[:, -1] - position_sequence[:, -2]
    return position_sequence[:, -1] + prev_vel + acc
